# Initial kernel scaffold; baseline (speedup 1.0000x reference)
import jax
import jax.numpy as jnp
from jax.experimental import pallas as pl


def kernel(x, edge_index, batch, W0, a_src0, a_dst0, b0, W1, a_src1, a_dst1, b1, W2, a_src2, a_dst2, b2, Wp, bp):
    raise NotImplementedError("write your pallas kernel here")


# speedup vs baseline: 39.3862x; 39.3862x over previous
import functools

import jax
import jax.numpy as jnp
from jax import lax
from jax.experimental import pallas as pl
from jax.experimental.pallas import tpu as pltpu
from jax.experimental.pallas import tpu_sc as plsc

NC, NS = 2, 16
NW = NC * NS
H, C, HC = 8, 16, 128
CHUNK = 128
BLK = 512


def _round_up(a, b):
    return (a + b - 1) // b * b


def _prep_first(xp, W, Msrc, Mdst, NP):
    def body(x_ref, w_ref, ms_ref, md_ref, h_ref, ts_ref, td_ref, mx_ref):
        i = pl.program_id(0)
        h = jnp.dot(x_ref[...], w_ref[...], preferred_element_type=jnp.float32)
        h_ref[...] = h
        ts = jnp.dot(h, ms_ref[...], preferred_element_type=jnp.float32)
        td = jnp.dot(h, md_ref[...], preferred_element_type=jnp.float32)
        ts_ref[...] = ts
        td_ref[...] = td
        m = jnp.concatenate(
            [jnp.max(ts, axis=0)[None, :], jnp.max(td, axis=0)[None, :],
             jnp.zeros((6, 16), jnp.float32)], axis=0)

        @pl.when(i == 0)
        def _():
            mx_ref[...] = m

        @pl.when(i != 0)
        def _():
            mx_ref[...] = jnp.maximum(mx_ref[...], m)

    grid = NP // BLK
    return pl.pallas_call(
        body,
        grid=(grid,),
        in_specs=[
            pl.BlockSpec((BLK, HC), lambda i: (i, 0)),
            pl.BlockSpec((HC, HC), lambda i: (0, 0)),
            pl.BlockSpec((HC, 16), lambda i: (0, 0)),
            pl.BlockSpec((HC, 16), lambda i: (0, 0)),
        ],
        out_specs=[
            pl.BlockSpec((BLK, HC), lambda i: (i, 0)),
            pl.BlockSpec((BLK, 16), lambda i: (i, 0)),
            pl.BlockSpec((BLK, 16), lambda i: (i, 0)),
            pl.BlockSpec((8, 16), lambda i: (0, 0)),
        ],
        out_shape=[
            jax.ShapeDtypeStruct((NP, HC), jnp.float32),
            jax.ShapeDtypeStruct((NP, 16), jnp.float32),
            jax.ShapeDtypeStruct((NP, 16), jnp.float32),
            jax.ShapeDtypeStruct((8, 16), jnp.float32),
        ],
    )(xp, W, Msrc, Mdst)


def _prep_next(part0, part1, bvec, W, Msrc, Mdst, NP, n_valid):
    def body(p0_ref, p1_ref, b_ref, w_ref, ms_ref, md_ref,
             h_ref, ts_ref, td_ref, mx_ref):
        i = pl.program_id(0)
        rows = i * BLK + lax.broadcasted_iota(jnp.int32, (BLK, 1), 0)
        act = jax.nn.relu(p0_ref[...] + p1_ref[...] + b_ref[...])
        act = jnp.where(rows < n_valid, act, 0.0)
        h = jnp.dot(act, w_ref[...], preferred_element_type=jnp.float32)
        h_ref[...] = h
        ts = jnp.dot(h, ms_ref[...], preferred_element_type=jnp.float32)
        td = jnp.dot(h, md_ref[...], preferred_element_type=jnp.float32)
        ts_ref[...] = ts
        td_ref[...] = td
        m = jnp.concatenate(
            [jnp.max(ts, axis=0)[None, :], jnp.max(td, axis=0)[None, :],
             jnp.zeros((6, 16), jnp.float32)], axis=0)

        @pl.when(i == 0)
        def _():
            mx_ref[...] = m

        @pl.when(i != 0)
        def _():
            mx_ref[...] = jnp.maximum(mx_ref[...], m)

    grid = NP // BLK
    return pl.pallas_call(
        body,
        grid=(grid,),
        in_specs=[
            pl.BlockSpec((BLK, HC), lambda i: (i, 0)),
            pl.BlockSpec((BLK, HC), lambda i: (i, 0)),
            pl.BlockSpec((1, HC), lambda i: (0, 0)),
            pl.BlockSpec((HC, HC), lambda i: (0, 0)),
            pl.BlockSpec((HC, 16), lambda i: (0, 0)),
            pl.BlockSpec((HC, 16), lambda i: (0, 0)),
        ],
        out_specs=[
            pl.BlockSpec((BLK, HC), lambda i: (i, 0)),
            pl.BlockSpec((BLK, 16), lambda i: (i, 0)),
            pl.BlockSpec((BLK, 16), lambda i: (i, 0)),
            pl.BlockSpec((8, 16), lambda i: (0, 0)),
        ],
        out_shape=[
            jax.ShapeDtypeStruct((NP, HC), jnp.float32),
            jax.ShapeDtypeStruct((NP, 16), jnp.float32),
            jax.ShapeDtypeStruct((NP, 16), jnp.float32),
            jax.ShapeDtypeStruct((8, 16), jnp.float32),
        ],
    )(part0, part1, bvec, W, Msrc, Mdst)


def _den_combine(den2, NP):
    def body(a_ref, b_ref, o_ref):
        o_ref[...] = a_ref[0] + b_ref[0]

    grid = NP // BLK
    return pl.pallas_call(
        body,
        grid=(grid,),
        in_specs=[
            pl.BlockSpec((1, BLK, 16), lambda i: (0, i, 0)),
            pl.BlockSpec((1, BLK, 16), lambda i: (1, i, 0)),
        ],
        out_specs=pl.BlockSpec((BLK, 16), lambda i: (i, 0)),
        out_shape=jax.ShapeDtypeStruct((NP, 16), jnp.float32),
    )(den2, den2)


def _final_proj(part0, part1, bvec, Wp, bp, NP, n_valid):
    def body(p0_ref, p1_ref, b_ref, wp_ref, bp_ref, o_ref):
        i = pl.program_id(0)
        rows = i * BLK + lax.broadcasted_iota(jnp.int32, (BLK, 1), 0)
        act = p0_ref[...] + p1_ref[...] + b_ref[...]
        act = jnp.where(rows < n_valid, act, 0.0)
        o_ref[...] = jnp.dot(act, wp_ref[...],
                             preferred_element_type=jnp.float32) + bp_ref[...]

    grid = NP // BLK
    return pl.pallas_call(
        body,
        grid=(grid,),
        in_specs=[
            pl.BlockSpec((BLK, HC), lambda i: (i, 0)),
            pl.BlockSpec((BLK, HC), lambda i: (i, 0)),
            pl.BlockSpec((1, HC), lambda i: (0, 0)),
            pl.BlockSpec((HC, 3), lambda i: (0, 0)),
            pl.BlockSpec((1, 3), lambda i: (0, 0)),
        ],
        out_specs=pl.BlockSpec((BLK, 3), lambda i: (i, 0)),
        out_shape=jax.ShapeDtypeStruct((n_valid, 3), jnp.float32),
    )(part0, part1, bvec, Wp, bp)


def _sc_edge_logits(tsrc, tdst, b16, srci, dsti, z16, NP, EP):
    per_tile = EP // NW
    nchunks = per_tile // CHUNK
    rows_pt = NP // NS
    mesh = plsc.VectorSubcoreMesh(core_axis_name="c", subcore_axis_name="s",
                                  num_cores=NC, num_subcores=NS)

    @functools.partial(
        pl.kernel, mesh=mesh,
        compiler_params=pltpu.CompilerParams(use_tc_tiling_on_sc=False),
        out_type=[jax.ShapeDtypeStruct((EP, 16), jnp.float32),
                  jax.ShapeDtypeStruct((NC, NP, 16), jnp.float32)],
        scratch_types=[
            pltpu.VMEM((CHUNK,), jnp.int32),
            pltpu.VMEM((CHUNK,), jnp.int32),
            pltpu.VMEM((CHUNK, 16), jnp.float32),
            pltpu.VMEM((CHUNK, 16), jnp.float32),
            pltpu.VMEM((CHUNK, 16), jnp.float32),
            pltpu.VMEM((16,), jnp.float32),
            pltpu.VMEM_SHARED((NP, 16), jnp.float32),
            pltpu.SemaphoreType.DMA,
            pltpu.SemaphoreType.DMA,
        ])
    def k(tsrc_h, tdst_h, b16_h, srci_h, dsti_h, z16_h, p_h, den_h,
          si_v, di_v, s_v, d_v, p_v, b_v, den_sp, sem1, sem2):
        cid = lax.axis_index("c")
        sid = lax.axis_index("s")
        wid = sid * NC + cid
        pltpu.sync_copy(z16_h.at[pl.ds(sid * rows_pt, rows_pt)],
                        den_sp.at[pl.ds(sid * rows_pt, rows_pt)])
        pltpu.sync_copy(b16_h, b_v)
        plsc.subcore_barrier()
        bb = b_v[...]
        base_w = wid * per_tile

        def chunk_body(i, carry):
            base = base_w + i * CHUNK
            pltpu.sync_copy(srci_h.at[pl.ds(base, CHUNK)], si_v)
            pltpu.sync_copy(dsti_h.at[pl.ds(base, CHUNK)], di_v)
            pltpu.async_copy(tsrc_h.at[si_v], s_v, sem1).wait()
            pltpu.async_copy(tdst_h.at[di_v], d_v, sem2).wait()

            def edge_body(e, c2):
                ev = s_v[e] + d_v[e]
                ev = jnp.maximum(ev, 0.0) + 0.2 * jnp.minimum(ev, 0.0)
                p_v[e] = jnp.exp(ev - bb)
                return c2

            lax.fori_loop(0, CHUNK, edge_body, 0, unroll=4)
            pltpu.sync_copy(p_v, den_sp.at[di_v], add=True)
            pltpu.sync_copy(p_v, p_h.at[pl.ds(base, CHUNK)])
            return carry

        lax.fori_loop(0, nchunks, chunk_body, 0)
        plsc.subcore_barrier()
        pltpu.sync_copy(den_sp.at[pl.ds(sid * rows_pt, rows_pt)],
                        den_h.at[cid, pl.ds(sid * rows_pt, rows_pt)])

    return k(tsrc, tdst, b16, srci, dsti, z16)


def _sc_messages(h, den, p, srci, dsti, z128, NP, EP):
    per_tile = EP // NW
    nchunks = per_tile // CHUNK
    rows_pt = NP // NS
    mesh = plsc.VectorSubcoreMesh(core_axis_name="c", subcore_axis_name="s",
                                  num_cores=NC, num_subcores=NS)

    @functools.partial(
        pl.kernel, mesh=mesh,
        compiler_params=pltpu.CompilerParams(use_tc_tiling_on_sc=False),
        out_type=jax.ShapeDtypeStruct((NC, NP, HC), jnp.float32),
        scratch_types=[
            pltpu.VMEM((CHUNK,), jnp.int32),
            pltpu.VMEM((CHUNK,), jnp.int32),
            pltpu.VMEM((CHUNK, 16), jnp.float32),
            pltpu.VMEM((CHUNK, 16), jnp.float32),
            pltpu.VMEM((CHUNK, HC), jnp.float32),
            pltpu.VMEM_SHARED((NP, HC), jnp.float32),
            pltpu.SemaphoreType.DMA,
            pltpu.SemaphoreType.DMA,
        ])
    def k(h_h, den_h, p_h, srci_h, dsti_h, z128_h, out_h,
          si_v, di_v, pa_v, dd_v, hs_v, out_sp, sem1, sem2):
        cid = lax.axis_index("c")
        sid = lax.axis_index("s")
        wid = sid * NC + cid
        pltpu.sync_copy(z128_h.at[pl.ds(sid * rows_pt, rows_pt)],
                        out_sp.at[pl.ds(sid * rows_pt, rows_pt)])
        plsc.subcore_barrier()
        base_w = wid * per_tile

        def chunk_body(i, carry):
            base = base_w + i * CHUNK
            pltpu.sync_copy(srci_h.at[pl.ds(base, CHUNK)], si_v)
            pltpu.sync_copy(dsti_h.at[pl.ds(base, CHUNK)], di_v)
            pltpu.async_copy(den_h.at[di_v], dd_v, sem1).wait()
            pltpu.sync_copy(p_h.at[pl.ds(base, CHUNK)], pa_v)
            pltpu.async_copy(h_h.at[si_v], hs_v, sem2).wait()

            def edge_body(e, c2):
                al = pa_v[e] / (dd_v[e] + 1e-16)
                for j in range(H):
                    sc = al[j]
                    hs_v[e, pl.ds(j * 16, 16)] = hs_v[e, pl.ds(j * 16, 16)] * sc
                return c2

            lax.fori_loop(0, CHUNK, edge_body, 0)
            pltpu.sync_copy(hs_v, out_sp.at[di_v], add=True)
            return carry

        lax.fori_loop(0, nchunks, chunk_body, 0)
        plsc.subcore_barrier()
        pltpu.sync_copy(out_sp.at[pl.ds(sid * rows_pt, rows_pt)],
                        out_h.at[cid, pl.ds(sid * rows_pt, rows_pt)])

    return k(h, den, p, srci, dsti, z128)


def _mmat(a):
    af = a.reshape(HC)
    ci = jnp.arange(HC, dtype=jnp.int32) // C
    jj = jnp.arange(16, dtype=jnp.int32) % H
    mask = (ci[:, None] == jj[None, :]).astype(jnp.float32)
    return af[:, None] * mask


def kernel(x, edge_index, batch, W0, a_src0, a_dst0, b0, W1, a_src1, a_dst1,
           b1, W2, a_src2, a_dst2, b2, Wp, bp):
    n = x.shape[0]
    e = edge_index.shape[1]
    NP = _round_up(n + 1, BLK)
    ep_raw = e + n
    per_tile = _round_up(-(-ep_raw // NW), CHUNK)
    EP = per_tile * NW

    loop = jnp.arange(n, dtype=jnp.int32)
    srci = jnp.concatenate(
        [edge_index[0], loop,
         jnp.zeros((EP - ep_raw,), jnp.int32)])
    dsti = jnp.concatenate(
        [edge_index[1], loop,
         jnp.full((EP - ep_raw,), n, jnp.int32)])

    xp = jnp.pad(x, ((0, NP - n), (0, 0)))
    z16 = jnp.zeros((NP, 16), jnp.float32)
    z128 = jnp.zeros((NP, HC), jnp.float32)

    Ws = [W0, W1, W2]
    Ms = [(_mmat(a_src0), _mmat(a_dst0)),
          (_mmat(a_src1), _mmat(a_dst1)),
          (_mmat(a_src2), _mmat(a_dst2))]
    bs = [b0.reshape(1, HC), b1.reshape(1, HC), b2.reshape(1, HC)]

    part0 = part1 = None
    for l in range(3):
        if l == 0:
            hmat, ts, td, mx = _prep_first(xp, Ws[0], Ms[0][0], Ms[0][1], NP)
        else:
            hmat, ts, td, mx = _prep_next(part0, part1, bs[l - 1], Ws[l],
                                          Ms[l][0], Ms[l][1], NP, n)
        b16 = mx[0] + mx[1]
        p, den2 = _sc_edge_logits(ts, td, b16, srci, dsti, z16, NP, EP)
        den = _den_combine(den2, NP)
        out2 = _sc_messages(hmat, den, p, srci, dsti, z128, NP, EP)
        part0, part1 = out2[0], out2[1]

    return _final_proj(part0, part1, bs[2], Wp, bp.reshape(1, 3), NP, n)


# speedup vs baseline: 63.9088x; 1.6226x over previous
import functools

import jax
import jax.numpy as jnp
from jax import lax
from jax.experimental import pallas as pl
from jax.experimental.pallas import tpu as pltpu
from jax.experimental.pallas import tpu_sc as plsc

NC, NS = 2, 16
NW = NC * NS
H, C, HC = 8, 16, 128
CHUNK = 128
BLK = 512


def _round_up(a, b):
    return (a + b - 1) // b * b


def _prep_first(xp, W, Msrc, Mdst, NP):
    def body(x_ref, w_ref, ms_ref, md_ref, h_ref, ts_ref, td_ref, mx_ref):
        i = pl.program_id(0)
        h = jnp.dot(x_ref[...], w_ref[...], preferred_element_type=jnp.float32)
        h_ref[...] = h
        ts = jnp.dot(h, ms_ref[...], preferred_element_type=jnp.float32)
        td = jnp.dot(h, md_ref[...], preferred_element_type=jnp.float32)
        ts_ref[...] = ts
        td_ref[...] = td
        m = jnp.concatenate(
            [jnp.max(ts, axis=0)[None, :], jnp.max(td, axis=0)[None, :],
             jnp.zeros((6, 16), jnp.float32)], axis=0)

        @pl.when(i == 0)
        def _():
            mx_ref[...] = m

        @pl.when(i != 0)
        def _():
            mx_ref[...] = jnp.maximum(mx_ref[...], m)

    grid = NP // BLK
    return pl.pallas_call(
        body,
        grid=(grid,),
        in_specs=[
            pl.BlockSpec((BLK, HC), lambda i: (i, 0)),
            pl.BlockSpec((HC, HC), lambda i: (0, 0)),
            pl.BlockSpec((HC, 16), lambda i: (0, 0)),
            pl.BlockSpec((HC, 16), lambda i: (0, 0)),
        ],
        out_specs=[
            pl.BlockSpec((BLK, HC), lambda i: (i, 0)),
            pl.BlockSpec((BLK, 16), lambda i: (i, 0)),
            pl.BlockSpec((BLK, 16), lambda i: (i, 0)),
            pl.BlockSpec((8, 16), lambda i: (0, 0)),
        ],
        out_shape=[
            jax.ShapeDtypeStruct((NP, HC), jnp.float32),
            jax.ShapeDtypeStruct((NP, 16), jnp.float32),
            jax.ShapeDtypeStruct((NP, 16), jnp.float32),
            jax.ShapeDtypeStruct((8, 16), jnp.float32),
        ],
    )(xp, W, Msrc, Mdst)


def _prep_next(part0, part1, bvec, W, Msrc, Mdst, NP, n_valid):
    def body(p0_ref, p1_ref, b_ref, w_ref, ms_ref, md_ref,
             h_ref, ts_ref, td_ref, mx_ref):
        i = pl.program_id(0)
        rows = i * BLK + lax.broadcasted_iota(jnp.int32, (BLK, 1), 0)
        act = jax.nn.relu(p0_ref[...] + p1_ref[...] + b_ref[...])
        act = jnp.where(rows < n_valid, act, 0.0)
        h = jnp.dot(act, w_ref[...], preferred_element_type=jnp.float32)
        h_ref[...] = h
        ts = jnp.dot(h, ms_ref[...], preferred_element_type=jnp.float32)
        td = jnp.dot(h, md_ref[...], preferred_element_type=jnp.float32)
        ts_ref[...] = ts
        td_ref[...] = td
        m = jnp.concatenate(
            [jnp.max(ts, axis=0)[None, :], jnp.max(td, axis=0)[None, :],
             jnp.zeros((6, 16), jnp.float32)], axis=0)

        @pl.when(i == 0)
        def _():
            mx_ref[...] = m

        @pl.when(i != 0)
        def _():
            mx_ref[...] = jnp.maximum(mx_ref[...], m)

    grid = NP // BLK
    return pl.pallas_call(
        body,
        grid=(grid,),
        in_specs=[
            pl.BlockSpec((BLK, HC), lambda i: (i, 0)),
            pl.BlockSpec((BLK, HC), lambda i: (i, 0)),
            pl.BlockSpec((1, HC), lambda i: (0, 0)),
            pl.BlockSpec((HC, HC), lambda i: (0, 0)),
            pl.BlockSpec((HC, 16), lambda i: (0, 0)),
            pl.BlockSpec((HC, 16), lambda i: (0, 0)),
        ],
        out_specs=[
            pl.BlockSpec((BLK, HC), lambda i: (i, 0)),
            pl.BlockSpec((BLK, 16), lambda i: (i, 0)),
            pl.BlockSpec((BLK, 16), lambda i: (i, 0)),
            pl.BlockSpec((8, 16), lambda i: (0, 0)),
        ],
        out_shape=[
            jax.ShapeDtypeStruct((NP, HC), jnp.float32),
            jax.ShapeDtypeStruct((NP, 16), jnp.float32),
            jax.ShapeDtypeStruct((NP, 16), jnp.float32),
            jax.ShapeDtypeStruct((8, 16), jnp.float32),
        ],
    )(part0, part1, bvec, W, Msrc, Mdst)


def _den_combine(den2, NP):
    def body(a_ref, b_ref, o_ref):
        o_ref[...] = 1.0 / (a_ref[0] + b_ref[0] + 1e-16)

    grid = NP // BLK
    return pl.pallas_call(
        body,
        grid=(grid,),
        in_specs=[
            pl.BlockSpec((1, BLK, 16), lambda i: (0, i, 0)),
            pl.BlockSpec((1, BLK, 16), lambda i: (1, i, 0)),
        ],
        out_specs=pl.BlockSpec((BLK, 16), lambda i: (i, 0)),
        out_shape=jax.ShapeDtypeStruct((NP, 16), jnp.float32),
    )(den2, den2)


def _final_proj(part0, part1, bvec, Wp, bp, NP, n_valid):
    def body(p0_ref, p1_ref, b_ref, wp_ref, bp_ref, o_ref):
        i = pl.program_id(0)
        rows = i * BLK + lax.broadcasted_iota(jnp.int32, (BLK, 1), 0)
        act = p0_ref[...] + p1_ref[...] + b_ref[...]
        act = jnp.where(rows < n_valid, act, 0.0)
        o_ref[...] = jnp.dot(act, wp_ref[...],
                             preferred_element_type=jnp.float32) + bp_ref[...]

    grid = NP // BLK
    return pl.pallas_call(
        body,
        grid=(grid,),
        in_specs=[
            pl.BlockSpec((BLK, HC), lambda i: (i, 0)),
            pl.BlockSpec((BLK, HC), lambda i: (i, 0)),
            pl.BlockSpec((1, HC), lambda i: (0, 0)),
            pl.BlockSpec((HC, 3), lambda i: (0, 0)),
            pl.BlockSpec((1, 3), lambda i: (0, 0)),
        ],
        out_specs=pl.BlockSpec((BLK, 3), lambda i: (i, 0)),
        out_shape=jax.ShapeDtypeStruct((n_valid, 3), jnp.float32),
    )(part0, part1, bvec, Wp, bp)


def _sc_edge_logits(tsrc, tdst, b16, srci, dsti, z16, NP, EP):
    per_tile = EP // NW
    nchunks = per_tile // CHUNK
    rows_pt = NP // NS
    mesh = plsc.VectorSubcoreMesh(core_axis_name="c", subcore_axis_name="s",
                                  num_cores=NC, num_subcores=NS)

    @functools.partial(
        pl.kernel, mesh=mesh,
        compiler_params=pltpu.CompilerParams(use_tc_tiling_on_sc=False),
        out_type=[jax.ShapeDtypeStruct((EP, 16), jnp.float32),
                  jax.ShapeDtypeStruct((NC, NP, 16), jnp.float32)],
        scratch_types=[
            pltpu.VMEM((2, CHUNK), jnp.int32),
            pltpu.VMEM((2, CHUNK), jnp.int32),
            pltpu.VMEM((2, CHUNK, 16), jnp.float32),
            pltpu.VMEM((2, CHUNK, 16), jnp.float32),
            pltpu.VMEM((2, CHUNK, 16), jnp.float32),
            pltpu.VMEM((16,), jnp.float32),
            pltpu.VMEM_SHARED((NP, 16), jnp.float32),
            pltpu.SemaphoreType.DMA,
            pltpu.SemaphoreType.DMA,
            pltpu.SemaphoreType.DMA,
        ])
    def k(tsrc_h, tdst_h, b16_h, srci_h, dsti_h, z16_h, p_h, den_h,
          si_v, di_v, s_v, d_v, p_v, b_v, den_sp, gsem, ssem, psem):
        cid = lax.axis_index("c")
        sid = lax.axis_index("s")
        wid = sid * NC + cid
        pltpu.sync_copy(z16_h.at[pl.ds(sid * rows_pt, rows_pt)],
                        den_sp.at[pl.ds(sid * rows_pt, rows_pt)])
        pltpu.sync_copy(b16_h, b_v)
        plsc.subcore_barrier()
        bb = b_v[...]
        base_w = wid * per_tile

        def fire(i, buf):
            base = base_w + i * CHUNK
            pltpu.sync_copy(srci_h.at[pl.ds(base, CHUNK)], si_v.at[buf])
            pltpu.sync_copy(dsti_h.at[pl.ds(base, CHUNK)], di_v.at[buf])
            pltpu.async_copy(tsrc_h.at[si_v.at[buf]], s_v.at[buf], gsem)
            pltpu.async_copy(tdst_h.at[di_v.at[buf]], d_v.at[buf], gsem)

        fire(0, 0)

        def chunk_body(i, carry):
            cur = lax.rem(i, 2)
            nxt = lax.rem(i + 1, 2)
            base = base_w + i * CHUNK

            @pl.when(i > 0)
            def _():
                pltpu.make_async_copy(
                    p_v.at[nxt], den_sp.at[di_v.at[nxt]], ssem).wait()
                pltpu.make_async_copy(
                    p_v.at[nxt], p_h.at[pl.ds(base - CHUNK, CHUNK)],
                    psem).wait()

            @pl.when(i + 1 < nchunks)
            def _():
                fire(i + 1, nxt)

            pltpu.make_async_copy(tsrc_h.at[si_v.at[cur]], s_v.at[cur],
                                  gsem).wait()
            pltpu.make_async_copy(tdst_h.at[di_v.at[cur]], d_v.at[cur],
                                  gsem).wait()

            def edge_body(e, c2):
                ev = s_v[cur, e] + d_v[cur, e]
                ev = jnp.maximum(ev, 0.0) + 0.2 * jnp.minimum(ev, 0.0)
                p_v[cur, e] = jnp.exp(ev - bb)
                return c2

            lax.fori_loop(0, CHUNK, edge_body, 0, unroll=4)
            pltpu.async_copy(p_v.at[cur], den_sp.at[di_v.at[cur]], ssem,
                             add=True)
            pltpu.async_copy(p_v.at[cur], p_h.at[pl.ds(base, CHUNK)], psem)
            return carry

        lax.fori_loop(0, nchunks, chunk_body, 0)
        last = lax.rem(nchunks - 1, 2)
        pltpu.make_async_copy(
            p_v.at[last], den_sp.at[di_v.at[last]], ssem).wait()
        pltpu.make_async_copy(
            p_v.at[last],
            p_h.at[pl.ds(base_w + (nchunks - 1) * CHUNK, CHUNK)], psem).wait()
        plsc.subcore_barrier()
        pltpu.sync_copy(den_sp.at[pl.ds(sid * rows_pt, rows_pt)],
                        den_h.at[cid, pl.ds(sid * rows_pt, rows_pt)])

    return k(tsrc, tdst, b16, srci, dsti, z16)


def _sc_messages(h, den, p, srci, dsti, z128, NP, EP):
    per_tile = EP // NW
    nchunks = per_tile // CHUNK
    rows_pt = NP // NS
    mesh = plsc.VectorSubcoreMesh(core_axis_name="c", subcore_axis_name="s",
                                  num_cores=NC, num_subcores=NS)

    @functools.partial(
        pl.kernel, mesh=mesh,
        compiler_params=pltpu.CompilerParams(use_tc_tiling_on_sc=False),
        out_type=jax.ShapeDtypeStruct((NC, NP, HC), jnp.float32),
        scratch_types=[
            pltpu.VMEM((2, CHUNK), jnp.int32),
            pltpu.VMEM((2, CHUNK), jnp.int32),
            pltpu.VMEM((2, CHUNK, 16), jnp.float32),
            pltpu.VMEM((2, CHUNK, 16), jnp.float32),
            pltpu.VMEM((2, CHUNK, HC), jnp.float32),
            pltpu.VMEM_SHARED((NP, HC), jnp.float32),
            pltpu.SemaphoreType.DMA,
            pltpu.SemaphoreType.DMA,
        ])
    def k(h_h, den_h, p_h, srci_h, dsti_h, z128_h, out_h,
          si_v, di_v, pa_v, dd_v, hs_v, out_sp, gsem, ssem):
        cid = lax.axis_index("c")
        sid = lax.axis_index("s")
        wid = sid * NC + cid
        pltpu.sync_copy(z128_h.at[pl.ds(sid * rows_pt, rows_pt)],
                        out_sp.at[pl.ds(sid * rows_pt, rows_pt)])
        plsc.subcore_barrier()
        base_w = wid * per_tile

        def fire(i, buf):
            base = base_w + i * CHUNK
            pltpu.sync_copy(srci_h.at[pl.ds(base, CHUNK)], si_v.at[buf])
            pltpu.sync_copy(dsti_h.at[pl.ds(base, CHUNK)], di_v.at[buf])
            pltpu.async_copy(den_h.at[di_v.at[buf]], dd_v.at[buf], gsem)
            pltpu.async_copy(p_h.at[pl.ds(base, CHUNK)], pa_v.at[buf], gsem)
            pltpu.async_copy(h_h.at[si_v.at[buf]], hs_v.at[buf], gsem)

        fire(0, 0)

        def chunk_body(i, carry):
            cur = lax.rem(i, 2)
            nxt = lax.rem(i + 1, 2)
            base = base_w + i * CHUNK

            @pl.when(i > 0)
            def _():
                pltpu.make_async_copy(
                    hs_v.at[nxt], out_sp.at[di_v.at[nxt]], ssem).wait()

            @pl.when(i + 1 < nchunks)
            def _():
                fire(i + 1, nxt)

            pltpu.make_async_copy(den_h.at[di_v.at[cur]], dd_v.at[cur],
                                  gsem).wait()
            pltpu.make_async_copy(p_h.at[pl.ds(base, CHUNK)], pa_v.at[cur],
                                  gsem).wait()
            pltpu.make_async_copy(h_h.at[si_v.at[cur]], hs_v.at[cur],
                                  gsem).wait()

            def edge_body(e, c2):
                al = pa_v[cur, e] * dd_v[cur, e]
                for j in range(H):
                    sc = al[j]
                    hs_v[cur, e, pl.ds(j * 16, 16)] = (
                        hs_v[cur, e, pl.ds(j * 16, 16)] * sc)
                return c2

            lax.fori_loop(0, CHUNK, edge_body, 0, unroll=2)
            pltpu.async_copy(hs_v.at[cur], out_sp.at[di_v.at[cur]], ssem,
                             add=True)
            return carry

        lax.fori_loop(0, nchunks, chunk_body, 0)
        last = lax.rem(nchunks - 1, 2)
        pltpu.make_async_copy(
            hs_v.at[last], out_sp.at[di_v.at[last]], ssem).wait()
        plsc.subcore_barrier()
        pltpu.sync_copy(out_sp.at[pl.ds(sid * rows_pt, rows_pt)],
                        out_h.at[cid, pl.ds(sid * rows_pt, rows_pt)])

    return k(h, den, p, srci, dsti, z128)


def _mmat(a):
    af = a.reshape(HC)
    ci = jnp.arange(HC, dtype=jnp.int32) // C
    jj = jnp.arange(16, dtype=jnp.int32) % H
    mask = (ci[:, None] == jj[None, :]).astype(jnp.float32)
    return af[:, None] * mask


def kernel(x, edge_index, batch, W0, a_src0, a_dst0, b0, W1, a_src1, a_dst1,
           b1, W2, a_src2, a_dst2, b2, Wp, bp):
    n = x.shape[0]
    e = edge_index.shape[1]
    NP = _round_up(n + 1, BLK)
    ep_raw = e + n
    per_tile = _round_up(-(-ep_raw // NW), CHUNK)
    EP = per_tile * NW

    loop = jnp.arange(n, dtype=jnp.int32)
    srci = jnp.concatenate(
        [edge_index[0], loop,
         jnp.zeros((EP - ep_raw,), jnp.int32)])
    dsti = jnp.concatenate(
        [edge_index[1], loop,
         jnp.full((EP - ep_raw,), n, jnp.int32)])

    xp = jnp.pad(x, ((0, NP - n), (0, 0)))
    z16 = jnp.zeros((NP, 16), jnp.float32)
    z128 = jnp.zeros((NP, HC), jnp.float32)

    Ws = [W0, W1, W2]
    Ms = [(_mmat(a_src0), _mmat(a_dst0)),
          (_mmat(a_src1), _mmat(a_dst1)),
          (_mmat(a_src2), _mmat(a_dst2))]
    bs = [b0.reshape(1, HC), b1.reshape(1, HC), b2.reshape(1, HC)]

    part0 = part1 = None
    for l in range(3):
        if l == 0:
            hmat, ts, td, mx = _prep_first(xp, Ws[0], Ms[0][0], Ms[0][1], NP)
        else:
            hmat, ts, td, mx = _prep_next(part0, part1, bs[l - 1], Ws[l],
                                          Ms[l][0], Ms[l][1], NP, n)
        b16 = mx[0] + mx[1]
        p, den2 = _sc_edge_logits(ts, td, b16, srci, dsti, z16, NP, EP)
        den = _den_combine(den2, NP)
        out2 = _sc_messages(hmat, den, p, srci, dsti, z128, NP, EP)
        part0, part1 = out2[0], out2[1]

    return _final_proj(part0, part1, bs[2], Wp, bp.reshape(1, 3), NP, n)


# speedup vs baseline: 89.2346x; 1.3963x over previous
import functools

import jax
import jax.numpy as jnp
from jax import lax
from jax.experimental import pallas as pl
from jax.experimental.pallas import tpu as pltpu
from jax.experimental.pallas import tpu_sc as plsc

NC, NS = 2, 16
NW = NC * NS
H, C, HC = 8, 16, 128
CHUNK = 128
BLK = 512


def _round_up(a, b):
    return (a + b - 1) // b * b


def _prep_first(xp, W, Msrc, Mdst, NP):
    def body(x_ref, w_ref, ms_ref, md_ref, h_ref, ts_ref, td_ref, mx_ref):
        i = pl.program_id(0)
        h = jnp.dot(x_ref[...], w_ref[...], preferred_element_type=jnp.float32)
        h_ref[...] = h
        ts = jnp.dot(h, ms_ref[...], preferred_element_type=jnp.float32)
        td = jnp.dot(h, md_ref[...], preferred_element_type=jnp.float32)
        ts_ref[...] = ts
        td_ref[...] = td
        m = jnp.concatenate(
            [jnp.max(ts, axis=0)[None, :], jnp.max(td, axis=0)[None, :],
             jnp.zeros((6, 16), jnp.float32)], axis=0)

        @pl.when(i == 0)
        def _():
            mx_ref[...] = m

        @pl.when(i != 0)
        def _():
            mx_ref[...] = jnp.maximum(mx_ref[...], m)

    grid = NP // BLK
    return pl.pallas_call(
        body,
        grid=(grid,),
        in_specs=[
            pl.BlockSpec((BLK, HC), lambda i: (i, 0)),
            pl.BlockSpec((HC, HC), lambda i: (0, 0)),
            pl.BlockSpec((HC, 16), lambda i: (0, 0)),
            pl.BlockSpec((HC, 16), lambda i: (0, 0)),
        ],
        out_specs=[
            pl.BlockSpec((BLK, HC), lambda i: (i, 0)),
            pl.BlockSpec((BLK, 16), lambda i: (i, 0)),
            pl.BlockSpec((BLK, 16), lambda i: (i, 0)),
            pl.BlockSpec((8, 16), lambda i: (0, 0)),
        ],
        out_shape=[
            jax.ShapeDtypeStruct((NP, HC), jnp.float32),
            jax.ShapeDtypeStruct((NP, 16), jnp.float32),
            jax.ShapeDtypeStruct((NP, 16), jnp.float32),
            jax.ShapeDtypeStruct((8, 16), jnp.float32),
        ],
    )(xp, W, Msrc, Mdst)


def _prep_next(part0, part1, bvec, W, Msrc, Mdst, NP, n_valid):
    def body(p0_ref, p1_ref, b_ref, w_ref, ms_ref, md_ref,
             h_ref, ts_ref, td_ref, mx_ref):
        i = pl.program_id(0)
        rows = i * BLK + lax.broadcasted_iota(jnp.int32, (BLK, 1), 0)
        act = jax.nn.relu(p0_ref[...] + p1_ref[...] + b_ref[...])
        act = jnp.where(rows < n_valid, act, 0.0)
        h = jnp.dot(act, w_ref[...], preferred_element_type=jnp.float32)
        h_ref[...] = h
        ts = jnp.dot(h, ms_ref[...], preferred_element_type=jnp.float32)
        td = jnp.dot(h, md_ref[...], preferred_element_type=jnp.float32)
        ts_ref[...] = ts
        td_ref[...] = td
        m = jnp.concatenate(
            [jnp.max(ts, axis=0)[None, :], jnp.max(td, axis=0)[None, :],
             jnp.zeros((6, 16), jnp.float32)], axis=0)

        @pl.when(i == 0)
        def _():
            mx_ref[...] = m

        @pl.when(i != 0)
        def _():
            mx_ref[...] = jnp.maximum(mx_ref[...], m)

    grid = NP // BLK
    return pl.pallas_call(
        body,
        grid=(grid,),
        in_specs=[
            pl.BlockSpec((BLK, HC), lambda i: (i, 0)),
            pl.BlockSpec((BLK, HC), lambda i: (i, 0)),
            pl.BlockSpec((1, HC), lambda i: (0, 0)),
            pl.BlockSpec((HC, HC), lambda i: (0, 0)),
            pl.BlockSpec((HC, 16), lambda i: (0, 0)),
            pl.BlockSpec((HC, 16), lambda i: (0, 0)),
        ],
        out_specs=[
            pl.BlockSpec((BLK, HC), lambda i: (i, 0)),
            pl.BlockSpec((BLK, 16), lambda i: (i, 0)),
            pl.BlockSpec((BLK, 16), lambda i: (i, 0)),
            pl.BlockSpec((8, 16), lambda i: (0, 0)),
        ],
        out_shape=[
            jax.ShapeDtypeStruct((NP, HC), jnp.float32),
            jax.ShapeDtypeStruct((NP, 16), jnp.float32),
            jax.ShapeDtypeStruct((NP, 16), jnp.float32),
            jax.ShapeDtypeStruct((8, 16), jnp.float32),
        ],
    )(part0, part1, bvec, W, Msrc, Mdst)


def _den_combine(den2, NP):
    def body(a_ref, b_ref, o_ref):
        o_ref[...] = 1.0 / (a_ref[0] + b_ref[0] + 1e-16)

    grid = NP // BLK
    return pl.pallas_call(
        body,
        grid=(grid,),
        in_specs=[
            pl.BlockSpec((1, BLK, 16), lambda i: (0, i, 0)),
            pl.BlockSpec((1, BLK, 16), lambda i: (1, i, 0)),
        ],
        out_specs=pl.BlockSpec((BLK, 16), lambda i: (i, 0)),
        out_shape=jax.ShapeDtypeStruct((NP, 16), jnp.float32),
    )(den2, den2)


def _final_proj(part0, part1, bvec, Wp, bp, NP, n_valid):
    def body(p0_ref, p1_ref, b_ref, wp_ref, bp_ref, o_ref):
        i = pl.program_id(0)
        rows = i * BLK + lax.broadcasted_iota(jnp.int32, (BLK, 1), 0)
        act = p0_ref[...] + p1_ref[...] + b_ref[...]
        act = jnp.where(rows < n_valid, act, 0.0)
        o_ref[...] = jnp.dot(act, wp_ref[...],
                             preferred_element_type=jnp.float32) + bp_ref[...]

    grid = NP // BLK
    return pl.pallas_call(
        body,
        grid=(grid,),
        in_specs=[
            pl.BlockSpec((BLK, HC), lambda i: (i, 0)),
            pl.BlockSpec((BLK, HC), lambda i: (i, 0)),
            pl.BlockSpec((1, HC), lambda i: (0, 0)),
            pl.BlockSpec((HC, 3), lambda i: (0, 0)),
            pl.BlockSpec((1, 3), lambda i: (0, 0)),
        ],
        out_specs=pl.BlockSpec((BLK, 3), lambda i: (i, 0)),
        out_shape=jax.ShapeDtypeStruct((n_valid, 3), jnp.float32),
    )(part0, part1, bvec, Wp, bp)


def _sc_edge_logits(tsrc, tdst, b16, srci, dsti, z16, NP, EP):
    per_tile = EP // NW
    nchunks = per_tile // CHUNK
    rows_pt = NP // NS
    mesh = plsc.VectorSubcoreMesh(core_axis_name="c", subcore_axis_name="s",
                                  num_cores=NC, num_subcores=NS)

    @functools.partial(
        pl.kernel, mesh=mesh,
        compiler_params=pltpu.CompilerParams(use_tc_tiling_on_sc=False),
        out_type=[jax.ShapeDtypeStruct((EP, 16), jnp.float32),
                  jax.ShapeDtypeStruct((NC, NP, 16), jnp.float32)],
        scratch_types=[
            pltpu.VMEM((2, CHUNK), jnp.int32),
            pltpu.VMEM((2, CHUNK), jnp.int32),
            pltpu.VMEM((2, CHUNK, 16), jnp.float32),
            pltpu.VMEM((2, CHUNK, 16), jnp.float32),
            pltpu.VMEM((2, CHUNK, 16), jnp.float32),
            pltpu.VMEM((16,), jnp.float32),
            pltpu.VMEM_SHARED((NP, 16), jnp.float32),
            pltpu.SemaphoreType.DMA,
            pltpu.SemaphoreType.DMA,
            pltpu.SemaphoreType.DMA,
        ])
    def k(tsrc_h, tdst_h, b16_h, srci_h, dsti_h, z16_h, p_h, den_h,
          si_v, di_v, s_v, d_v, p_v, b_v, den_sp, gsem, ssem, psem):
        cid = lax.axis_index("c")
        sid = lax.axis_index("s")
        wid = sid * NC + cid
        pltpu.sync_copy(z16_h.at[pl.ds(sid * rows_pt, rows_pt)],
                        den_sp.at[pl.ds(sid * rows_pt, rows_pt)])
        pltpu.sync_copy(b16_h, b_v)
        plsc.subcore_barrier()
        bb = b_v[...]
        base_w = wid * per_tile

        def fire(i, buf):
            base = base_w + i * CHUNK
            pltpu.sync_copy(srci_h.at[pl.ds(base, CHUNK)], si_v.at[buf])
            pltpu.sync_copy(dsti_h.at[pl.ds(base, CHUNK)], di_v.at[buf])
            pltpu.async_copy(tsrc_h.at[si_v.at[buf]], s_v.at[buf], gsem)
            pltpu.async_copy(tdst_h.at[di_v.at[buf]], d_v.at[buf], gsem)

        fire(0, 0)

        def chunk_body(i, carry):
            cur = lax.rem(i, 2)
            nxt = lax.rem(i + 1, 2)
            base = base_w + i * CHUNK

            @pl.when(i > 0)
            def _():
                pltpu.make_async_copy(
                    p_v.at[nxt], den_sp.at[di_v.at[nxt]], ssem).wait()
                pltpu.make_async_copy(
                    p_v.at[nxt], p_h.at[pl.ds(base - CHUNK, CHUNK)],
                    psem).wait()

            @pl.when(i + 1 < nchunks)
            def _():
                fire(i + 1, nxt)

            pltpu.make_async_copy(tsrc_h.at[si_v.at[cur]], s_v.at[cur],
                                  gsem).wait()
            pltpu.make_async_copy(tdst_h.at[di_v.at[cur]], d_v.at[cur],
                                  gsem).wait()

            @plsc.parallel_loop(0, CHUNK, unroll=8)
            def edge_body(e):
                ev = s_v[cur, e] + d_v[cur, e]
                ev = jnp.maximum(ev, 0.0) + 0.2 * jnp.minimum(ev, 0.0)
                p_v[cur, e] = jnp.exp(ev - bb)
            pltpu.async_copy(p_v.at[cur], den_sp.at[di_v.at[cur]], ssem,
                             add=True)
            pltpu.async_copy(p_v.at[cur], p_h.at[pl.ds(base, CHUNK)], psem)
            return carry

        lax.fori_loop(0, nchunks, chunk_body, 0)
        last = lax.rem(nchunks - 1, 2)
        pltpu.make_async_copy(
            p_v.at[last], den_sp.at[di_v.at[last]], ssem).wait()
        pltpu.make_async_copy(
            p_v.at[last],
            p_h.at[pl.ds(base_w + (nchunks - 1) * CHUNK, CHUNK)], psem).wait()
        plsc.subcore_barrier()
        pltpu.sync_copy(den_sp.at[pl.ds(sid * rows_pt, rows_pt)],
                        den_h.at[cid, pl.ds(sid * rows_pt, rows_pt)])

    return k(tsrc, tdst, b16, srci, dsti, z16)


def _sc_messages(h, den, p, srci, dsti, z128, NP, EP):
    per_tile = EP // NW
    nchunks = per_tile // CHUNK
    rows_pt = NP // NS
    mesh = plsc.VectorSubcoreMesh(core_axis_name="c", subcore_axis_name="s",
                                  num_cores=NC, num_subcores=NS)

    @functools.partial(
        pl.kernel, mesh=mesh,
        compiler_params=pltpu.CompilerParams(use_tc_tiling_on_sc=False),
        out_type=jax.ShapeDtypeStruct((NC, NP, HC), jnp.float32),
        scratch_types=[
            pltpu.VMEM((2, CHUNK), jnp.int32),
            pltpu.VMEM((2, CHUNK), jnp.int32),
            pltpu.VMEM((2, CHUNK, 16), jnp.float32),
            pltpu.VMEM((2, CHUNK, 16), jnp.float32),
            pltpu.VMEM((2, CHUNK, HC), jnp.float32),
            pltpu.VMEM_SHARED((NP, HC), jnp.float32),
            pltpu.SemaphoreType.DMA,
            pltpu.SemaphoreType.DMA,
        ])
    def k(h_h, den_h, p_h, srci_h, dsti_h, z128_h, out_h,
          si_v, di_v, pa_v, dd_v, hs_v, out_sp, gsem, ssem):
        cid = lax.axis_index("c")
        sid = lax.axis_index("s")
        wid = sid * NC + cid
        pltpu.sync_copy(z128_h.at[pl.ds(sid * rows_pt, rows_pt)],
                        out_sp.at[pl.ds(sid * rows_pt, rows_pt)])
        plsc.subcore_barrier()
        base_w = wid * per_tile

        def fire(i, buf):
            base = base_w + i * CHUNK
            pltpu.sync_copy(srci_h.at[pl.ds(base, CHUNK)], si_v.at[buf])
            pltpu.sync_copy(dsti_h.at[pl.ds(base, CHUNK)], di_v.at[buf])
            pltpu.async_copy(den_h.at[di_v.at[buf]], dd_v.at[buf], gsem)
            pltpu.async_copy(p_h.at[pl.ds(base, CHUNK)], pa_v.at[buf], gsem)
            pltpu.async_copy(h_h.at[si_v.at[buf]], hs_v.at[buf], gsem)

        fire(0, 0)

        def chunk_body(i, carry):
            cur = lax.rem(i, 2)
            nxt = lax.rem(i + 1, 2)
            base = base_w + i * CHUNK

            @pl.when(i > 0)
            def _():
                pltpu.make_async_copy(
                    hs_v.at[nxt], out_sp.at[di_v.at[nxt]], ssem).wait()

            @pl.when(i + 1 < nchunks)
            def _():
                fire(i + 1, nxt)

            pltpu.make_async_copy(den_h.at[di_v.at[cur]], dd_v.at[cur],
                                  gsem).wait()
            pltpu.make_async_copy(p_h.at[pl.ds(base, CHUNK)], pa_v.at[cur],
                                  gsem).wait()
            pltpu.make_async_copy(h_h.at[si_v.at[cur]], hs_v.at[cur],
                                  gsem).wait()

            @plsc.parallel_loop(0, CHUNK, unroll=4)
            def edge_body(e):
                al = pa_v[cur, e] * dd_v[cur, e]
                for j in range(H):
                    sc = al[j]
                    hs_v[cur, e, pl.ds(j * 16, 16)] = (
                        hs_v[cur, e, pl.ds(j * 16, 16)] * sc)
            pltpu.async_copy(hs_v.at[cur], out_sp.at[di_v.at[cur]], ssem,
                             add=True)
            return carry

        lax.fori_loop(0, nchunks, chunk_body, 0)
        last = lax.rem(nchunks - 1, 2)
        pltpu.make_async_copy(
            hs_v.at[last], out_sp.at[di_v.at[last]], ssem).wait()
        plsc.subcore_barrier()
        pltpu.sync_copy(out_sp.at[pl.ds(sid * rows_pt, rows_pt)],
                        out_h.at[cid, pl.ds(sid * rows_pt, rows_pt)])

    return k(h, den, p, srci, dsti, z128)


def _mmat(a):
    af = a.reshape(HC)
    ci = jnp.arange(HC, dtype=jnp.int32) // C
    jj = jnp.arange(16, dtype=jnp.int32) % H
    mask = (ci[:, None] == jj[None, :]).astype(jnp.float32)
    return af[:, None] * mask


def kernel(x, edge_index, batch, W0, a_src0, a_dst0, b0, W1, a_src1, a_dst1,
           b1, W2, a_src2, a_dst2, b2, Wp, bp):
    n = x.shape[0]
    e = edge_index.shape[1]
    NP = _round_up(n + 1, BLK)
    ep_raw = e + n
    per_tile = _round_up(-(-ep_raw // NW), CHUNK)
    EP = per_tile * NW

    loop = jnp.arange(n, dtype=jnp.int32)
    srci = jnp.concatenate(
        [edge_index[0], loop,
         jnp.zeros((EP - ep_raw,), jnp.int32)])
    dsti = jnp.concatenate(
        [edge_index[1], loop,
         jnp.full((EP - ep_raw,), n, jnp.int32)])

    xp = jnp.pad(x, ((0, NP - n), (0, 0)))
    z16 = jnp.zeros((NP, 16), jnp.float32)
    z128 = jnp.zeros((NP, HC), jnp.float32)

    Ws = [W0, W1, W2]
    Ms = [(_mmat(a_src0), _mmat(a_dst0)),
          (_mmat(a_src1), _mmat(a_dst1)),
          (_mmat(a_src2), _mmat(a_dst2))]
    bs = [b0.reshape(1, HC), b1.reshape(1, HC), b2.reshape(1, HC)]

    part0 = part1 = None
    for l in range(3):
        if l == 0:
            hmat, ts, td, mx = _prep_first(xp, Ws[0], Ms[0][0], Ms[0][1], NP)
        else:
            hmat, ts, td, mx = _prep_next(part0, part1, bs[l - 1], Ws[l],
                                          Ms[l][0], Ms[l][1], NP, n)
        b16 = mx[0] + mx[1]
        p, den2 = _sc_edge_logits(ts, td, b16, srci, dsti, z16, NP, EP)
        den = _den_combine(den2, NP)
        out2 = _sc_messages(hmat, den, p, srci, dsti, z128, NP, EP)
        part0, part1 = out2[0], out2[1]

    return _final_proj(part0, part1, bs[2], Wp, bp.reshape(1, 3), NP, n)


# speedup vs baseline: 105.5155x; 1.1825x over previous
import functools

import jax
import jax.numpy as jnp
from jax import lax
from jax.experimental import pallas as pl
from jax.experimental.pallas import tpu as pltpu
from jax.experimental.pallas import tpu_sc as plsc

NC, NS = 2, 16
NW = NC * NS
H, C, HC = 8, 16, 128
CHUNK = 128
BLK = 512


def _round_up(a, b):
    return (a + b - 1) // b * b


def _prep_first(xp, W, Msrc, Mdst, NP):
    def body(x_ref, w_ref, ms_ref, md_ref, h_ref, ts_ref, td_ref, mx_ref):
        i = pl.program_id(0)
        h = jnp.dot(x_ref[...], w_ref[...], preferred_element_type=jnp.float32)
        h_ref[...] = h
        ts = jnp.dot(h, ms_ref[...], preferred_element_type=jnp.float32)
        td = jnp.dot(h, md_ref[...], preferred_element_type=jnp.float32)
        ts_ref[...] = ts
        td_ref[...] = td
        m = jnp.concatenate(
            [jnp.max(ts, axis=0)[None, :], jnp.max(td, axis=0)[None, :],
             jnp.zeros((6, 16), jnp.float32)], axis=0)

        @pl.when(i == 0)
        def _():
            mx_ref[...] = m

        @pl.when(i != 0)
        def _():
            mx_ref[...] = jnp.maximum(mx_ref[...], m)

    grid = NP // BLK
    return pl.pallas_call(
        body,
        grid=(grid,),
        in_specs=[
            pl.BlockSpec((BLK, HC), lambda i: (i, 0)),
            pl.BlockSpec((HC, HC), lambda i: (0, 0)),
            pl.BlockSpec((HC, 16), lambda i: (0, 0)),
            pl.BlockSpec((HC, 16), lambda i: (0, 0)),
        ],
        out_specs=[
            pl.BlockSpec((BLK, HC), lambda i: (i, 0)),
            pl.BlockSpec((BLK, 16), lambda i: (i, 0)),
            pl.BlockSpec((BLK, 16), lambda i: (i, 0)),
            pl.BlockSpec((8, 16), lambda i: (0, 0)),
        ],
        out_shape=[
            jax.ShapeDtypeStruct((NP, HC), jnp.float32),
            jax.ShapeDtypeStruct((NP, 16), jnp.float32),
            jax.ShapeDtypeStruct((NP, 16), jnp.float32),
            jax.ShapeDtypeStruct((8, 16), jnp.float32),
        ],
    )(xp, W, Msrc, Mdst)


def _prep_next(part0, part1, bvec, W, Msrc, Mdst, NP, n_valid):
    def body(p0_ref, p1_ref, b_ref, w_ref, ms_ref, md_ref,
             h_ref, ts_ref, td_ref, mx_ref):
        i = pl.program_id(0)
        rows = i * BLK + lax.broadcasted_iota(jnp.int32, (BLK, 1), 0)
        act = jax.nn.relu(p0_ref[...] + p1_ref[...] + b_ref[...])
        act = jnp.where(rows < n_valid, act, 0.0)
        h = jnp.dot(act, w_ref[...], preferred_element_type=jnp.float32)
        h_ref[...] = h
        ts = jnp.dot(h, ms_ref[...], preferred_element_type=jnp.float32)
        td = jnp.dot(h, md_ref[...], preferred_element_type=jnp.float32)
        ts_ref[...] = ts
        td_ref[...] = td
        m = jnp.concatenate(
            [jnp.max(ts, axis=0)[None, :], jnp.max(td, axis=0)[None, :],
             jnp.zeros((6, 16), jnp.float32)], axis=0)

        @pl.when(i == 0)
        def _():
            mx_ref[...] = m

        @pl.when(i != 0)
        def _():
            mx_ref[...] = jnp.maximum(mx_ref[...], m)

    grid = NP // BLK
    return pl.pallas_call(
        body,
        grid=(grid,),
        in_specs=[
            pl.BlockSpec((BLK, HC), lambda i: (i, 0)),
            pl.BlockSpec((BLK, HC), lambda i: (i, 0)),
            pl.BlockSpec((1, HC), lambda i: (0, 0)),
            pl.BlockSpec((HC, HC), lambda i: (0, 0)),
            pl.BlockSpec((HC, 16), lambda i: (0, 0)),
            pl.BlockSpec((HC, 16), lambda i: (0, 0)),
        ],
        out_specs=[
            pl.BlockSpec((BLK, HC), lambda i: (i, 0)),
            pl.BlockSpec((BLK, 16), lambda i: (i, 0)),
            pl.BlockSpec((BLK, 16), lambda i: (i, 0)),
            pl.BlockSpec((8, 16), lambda i: (0, 0)),
        ],
        out_shape=[
            jax.ShapeDtypeStruct((NP, HC), jnp.float32),
            jax.ShapeDtypeStruct((NP, 16), jnp.float32),
            jax.ShapeDtypeStruct((NP, 16), jnp.float32),
            jax.ShapeDtypeStruct((8, 16), jnp.float32),
        ],
    )(part0, part1, bvec, W, Msrc, Mdst)


def _den_combine(den2, NP):
    def body(a_ref, b_ref, o_ref):
        o_ref[...] = 1.0 / (a_ref[0] + b_ref[0] + 1e-16)

    grid = NP // BLK
    return pl.pallas_call(
        body,
        grid=(grid,),
        in_specs=[
            pl.BlockSpec((1, BLK, 16), lambda i: (0, i, 0)),
            pl.BlockSpec((1, BLK, 16), lambda i: (1, i, 0)),
        ],
        out_specs=pl.BlockSpec((BLK, 16), lambda i: (i, 0)),
        out_shape=jax.ShapeDtypeStruct((NP, 16), jnp.float32),
    )(den2, den2)


def _final_proj(part0, part1, bvec, Wp, bp, NP, n_valid):
    def body(p0_ref, p1_ref, b_ref, wp_ref, bp_ref, o_ref):
        i = pl.program_id(0)
        rows = i * BLK + lax.broadcasted_iota(jnp.int32, (BLK, 1), 0)
        act = p0_ref[...] + p1_ref[...] + b_ref[...]
        act = jnp.where(rows < n_valid, act, 0.0)
        o_ref[...] = jnp.dot(act, wp_ref[...],
                             preferred_element_type=jnp.float32) + bp_ref[...]

    grid = NP // BLK
    return pl.pallas_call(
        body,
        grid=(grid,),
        in_specs=[
            pl.BlockSpec((BLK, HC), lambda i: (i, 0)),
            pl.BlockSpec((BLK, HC), lambda i: (i, 0)),
            pl.BlockSpec((1, HC), lambda i: (0, 0)),
            pl.BlockSpec((HC, 3), lambda i: (0, 0)),
            pl.BlockSpec((1, 3), lambda i: (0, 0)),
        ],
        out_specs=pl.BlockSpec((BLK, 3), lambda i: (i, 0)),
        out_shape=jax.ShapeDtypeStruct((n_valid, 3), jnp.float32),
    )(part0, part1, bvec, Wp, bp)


def _sc_edge_logits(tsrc, tdst, b16, srci, dsti, z16, NP, EP):
    per_tile = EP // NW
    nchunks = per_tile // CHUNK
    rows_pt = NP // NS
    mesh = plsc.VectorSubcoreMesh(core_axis_name="c", subcore_axis_name="s",
                                  num_cores=NC, num_subcores=NS)

    @functools.partial(
        pl.kernel, mesh=mesh,
        compiler_params=pltpu.CompilerParams(use_tc_tiling_on_sc=False),
        out_type=[jax.ShapeDtypeStruct((EP, 16), jnp.float32),
                  jax.ShapeDtypeStruct((NC, NP, 16), jnp.float32)],
        scratch_types=[
            pltpu.VMEM((nchunks, CHUNK), jnp.int32),
            pltpu.VMEM((nchunks, CHUNK), jnp.int32),
            pltpu.VMEM((2, CHUNK, 16), jnp.float32),
            pltpu.VMEM((2, CHUNK, 16), jnp.float32),
            pltpu.VMEM((2, CHUNK, 16), jnp.float32),
            pltpu.VMEM((16,), jnp.float32),
            pltpu.VMEM_SHARED((NP, 16), jnp.float32),
            pltpu.SemaphoreType.DMA,
            pltpu.SemaphoreType.DMA,
            pltpu.SemaphoreType.DMA,
        ])
    def k(tsrc_h, tdst_h, b16_h, srci_h, dsti_h, z16_h, p_h, den_h,
          si_v, di_v, s_v, d_v, p_v, b_v, den_sp, gsem, ssem, psem):
        cid = lax.axis_index("c")
        sid = lax.axis_index("s")
        wid = sid * NC + cid
        pltpu.sync_copy(z16_h.at[pl.ds(sid * rows_pt, rows_pt)],
                        den_sp.at[pl.ds(sid * rows_pt, rows_pt)])
        pltpu.sync_copy(b16_h, b_v)
        pltpu.sync_copy(srci_h.at[pl.ds(wid * nchunks, nchunks)], si_v)
        pltpu.sync_copy(dsti_h.at[pl.ds(wid * nchunks, nchunks)], di_v)
        plsc.subcore_barrier()
        bb = b_v[...]
        base_w = wid * per_tile

        def fire(i, buf):
            pltpu.async_copy(tsrc_h.at[si_v.at[i]], s_v.at[buf], gsem)
            pltpu.async_copy(tdst_h.at[di_v.at[i]], d_v.at[buf], gsem)

        fire(0, 0)

        def chunk_body(i, carry):
            cur = lax.rem(i, 2)
            nxt = lax.rem(i + 1, 2)
            base = base_w + i * CHUNK

            @pl.when(i > 0)
            def _():
                pltpu.make_async_copy(
                    p_v.at[nxt], den_sp.at[di_v.at[i - 1]], ssem).wait()
                pltpu.make_async_copy(
                    p_v.at[nxt], p_h.at[pl.ds(base - CHUNK, CHUNK)],
                    psem).wait()

            @pl.when(i + 1 < nchunks)
            def _():
                fire(i + 1, nxt)

            pltpu.make_async_copy(tsrc_h.at[si_v.at[i]], s_v.at[cur],
                                  gsem).wait()
            pltpu.make_async_copy(tdst_h.at[di_v.at[i]], d_v.at[cur],
                                  gsem).wait()

            @plsc.parallel_loop(0, CHUNK, unroll=8)
            def edge_body(e):
                ev = s_v[cur, e] + d_v[cur, e]
                ev = jnp.maximum(ev, 0.0) + 0.2 * jnp.minimum(ev, 0.0)
                p_v[cur, e] = jnp.exp(ev - bb)
            pltpu.async_copy(p_v.at[cur], den_sp.at[di_v.at[i]], ssem,
                             add=True)
            pltpu.async_copy(p_v.at[cur], p_h.at[pl.ds(base, CHUNK)], psem)
            return carry

        lax.fori_loop(0, nchunks, chunk_body, 0)
        last = lax.rem(nchunks - 1, 2)
        pltpu.make_async_copy(
            p_v.at[last], den_sp.at[di_v.at[nchunks - 1]], ssem).wait()
        pltpu.make_async_copy(
            p_v.at[last],
            p_h.at[pl.ds(base_w + (nchunks - 1) * CHUNK, CHUNK)], psem).wait()
        plsc.subcore_barrier()
        pltpu.sync_copy(den_sp.at[pl.ds(sid * rows_pt, rows_pt)],
                        den_h.at[cid, pl.ds(sid * rows_pt, rows_pt)])

    return k(tsrc, tdst, b16, srci, dsti, z16)


def _sc_messages(h, den, p, srci, dsti, z128, NP, EP):
    per_tile = EP // NW
    nchunks = per_tile // CHUNK
    rows_pt = NP // NS
    mesh = plsc.VectorSubcoreMesh(core_axis_name="c", subcore_axis_name="s",
                                  num_cores=NC, num_subcores=NS)

    @functools.partial(
        pl.kernel, mesh=mesh,
        compiler_params=pltpu.CompilerParams(use_tc_tiling_on_sc=False),
        out_type=jax.ShapeDtypeStruct((NC, NP, HC), jnp.float32),
        scratch_types=[
            pltpu.VMEM((3, CHUNK), jnp.int32),
            pltpu.VMEM((3, CHUNK), jnp.int32),
            pltpu.VMEM((2, CHUNK, 16), jnp.float32),
            pltpu.VMEM((2, CHUNK, 16), jnp.float32),
            pltpu.VMEM((2, CHUNK, HC), jnp.float32),
            pltpu.VMEM_SHARED((NP, HC), jnp.float32),
            pltpu.SemaphoreType.DMA,
            pltpu.SemaphoreType.DMA,
            pltpu.SemaphoreType.DMA,
        ])
    def k(h_h, den_h, p_h, srci_h, dsti_h, z128_h, out_h,
          si_v, di_v, pa_v, dd_v, hs_v, out_sp, gsem, ssem, isem):
        cid = lax.axis_index("c")
        sid = lax.axis_index("s")
        wid = sid * NC + cid
        pltpu.sync_copy(z128_h.at[pl.ds(sid * rows_pt, rows_pt)],
                        out_sp.at[pl.ds(sid * rows_pt, rows_pt)])
        plsc.subcore_barrier()
        base_w = wid * nchunks

        def load_idx(i, slot):
            pltpu.async_copy(srci_h.at[pl.ds(base_w + i, 1)],
                             si_v.at[pl.ds(slot, 1)], isem)
            pltpu.async_copy(dsti_h.at[pl.ds(base_w + i, 1)],
                             di_v.at[pl.ds(slot, 1)], isem)

        def fire(i, buf, slot):
            pltpu.make_async_copy(srci_h.at[pl.ds(base_w + i, 1)],
                                  si_v.at[pl.ds(slot, 1)], isem).wait()
            pltpu.make_async_copy(dsti_h.at[pl.ds(base_w + i, 1)],
                                  di_v.at[pl.ds(slot, 1)], isem).wait()
            base = (base_w + i) * CHUNK
            pltpu.async_copy(den_h.at[di_v.at[slot]], dd_v.at[buf], gsem)
            pltpu.async_copy(p_h.at[pl.ds(base, CHUNK)], pa_v.at[buf], gsem)
            pltpu.async_copy(h_h.at[si_v.at[slot]], hs_v.at[buf], gsem)

        load_idx(0, 0)
        fire(0, 0, 0)
        load_idx(1, 1)

        def chunk_body(i, carry):
            cur = lax.rem(i, 2)
            nxt = lax.rem(i + 1, 2)
            slot = lax.rem(i, 3)
            slot1 = lax.rem(i + 1, 3)
            slot2 = lax.rem(i + 2, 3)
            base = (base_w + i) * CHUNK

            @pl.when(i > 0)
            def _():
                pltpu.make_async_copy(
                    hs_v.at[nxt], out_sp.at[di_v.at[slot2]], ssem).wait()

            @pl.when(i + 1 < nchunks)
            def _():
                fire(i + 1, nxt, slot1)

            @pl.when(i + 2 < nchunks)
            def _():
                load_idx(i + 2, slot2)

            pltpu.make_async_copy(den_h.at[di_v.at[slot]], dd_v.at[cur],
                                  gsem).wait()
            pltpu.make_async_copy(p_h.at[pl.ds(base, CHUNK)], pa_v.at[cur],
                                  gsem).wait()
            pltpu.make_async_copy(h_h.at[si_v.at[slot]], hs_v.at[cur],
                                  gsem).wait()

            @plsc.parallel_loop(0, CHUNK, unroll=4)
            def edge_body(e):
                al = pa_v[cur, e] * dd_v[cur, e]
                for j in range(H):
                    sc = al[j]
                    hs_v[cur, e, pl.ds(j * 16, 16)] = (
                        hs_v[cur, e, pl.ds(j * 16, 16)] * sc)
            pltpu.async_copy(hs_v.at[cur], out_sp.at[di_v.at[slot]], ssem,
                             add=True)
            return carry

        lax.fori_loop(0, nchunks, chunk_body, 0)
        last = lax.rem(nchunks - 1, 2)
        pltpu.make_async_copy(
            hs_v.at[last],
            out_sp.at[di_v.at[lax.rem(nchunks - 1, 3)]], ssem).wait()
        plsc.subcore_barrier()
        pltpu.sync_copy(out_sp.at[pl.ds(sid * rows_pt, rows_pt)],
                        out_h.at[cid, pl.ds(sid * rows_pt, rows_pt)])

    return k(h, den, p, srci, dsti, z128)


def _mmat(a):
    af = a.reshape(HC)
    ci = jnp.arange(HC, dtype=jnp.int32) // C
    jj = jnp.arange(16, dtype=jnp.int32) % H
    mask = (ci[:, None] == jj[None, :]).astype(jnp.float32)
    return af[:, None] * mask


def kernel(x, edge_index, batch, W0, a_src0, a_dst0, b0, W1, a_src1, a_dst1,
           b1, W2, a_src2, a_dst2, b2, Wp, bp):
    n = x.shape[0]
    e = edge_index.shape[1]
    NP = _round_up(n + 1, BLK)
    ep_raw = e + n
    per_tile = _round_up(-(-ep_raw // NW), CHUNK)
    EP = per_tile * NW

    loop = jnp.arange(n, dtype=jnp.int32)
    srci = jnp.concatenate(
        [edge_index[0], loop,
         jnp.zeros((EP - ep_raw,), jnp.int32)]).reshape(EP // CHUNK, CHUNK)
    dsti = jnp.concatenate(
        [edge_index[1], loop,
         jnp.full((EP - ep_raw,), n, jnp.int32)]).reshape(EP // CHUNK, CHUNK)

    xp = jnp.pad(x, ((0, NP - n), (0, 0)))
    z16 = jnp.zeros((NP, 16), jnp.float32)
    z128 = jnp.zeros((NP, HC), jnp.float32)

    Ws = [W0, W1, W2]
    Ms = [(_mmat(a_src0), _mmat(a_dst0)),
          (_mmat(a_src1), _mmat(a_dst1)),
          (_mmat(a_src2), _mmat(a_dst2))]
    bs = [b0.reshape(1, HC), b1.reshape(1, HC), b2.reshape(1, HC)]

    part0 = part1 = None
    for l in range(3):
        if l == 0:
            hmat, ts, td, mx = _prep_first(xp, Ws[0], Ms[0][0], Ms[0][1], NP)
        else:
            hmat, ts, td, mx = _prep_next(part0, part1, bs[l - 1], Ws[l],
                                          Ms[l][0], Ms[l][1], NP, n)
        b16 = mx[0] + mx[1]
        p, den2 = _sc_edge_logits(ts, td, b16, srci, dsti, z16, NP, EP)
        den = _den_combine(den2, NP)
        out2 = _sc_messages(hmat, den, p, srci, dsti, z128, NP, EP)
        part0, part1 = out2[0], out2[1]

    return _final_proj(part0, part1, bs[2], Wp, bp.reshape(1, 3), NP, n)


# speedup vs baseline: 107.4500x; 1.0183x over previous
import functools

import jax
import jax.numpy as jnp
from jax import lax
from jax.experimental import pallas as pl
from jax.experimental.pallas import tpu as pltpu
from jax.experimental.pallas import tpu_sc as plsc

NC, NS = 2, 16
NW = NC * NS
H, C, HC = 8, 16, 128
CHUNK = 128
BLK = 512


def _round_up(a, b):
    return (a + b - 1) // b * b


def _prep_first(xp, W, Msrc, Mdst, NP):
    def body(x_ref, w_ref, ms_ref, md_ref, h_ref, ts_ref, td_ref, mx_ref):
        i = pl.program_id(0)
        h = jnp.dot(x_ref[...], w_ref[...], preferred_element_type=jnp.float32)
        h_ref[...] = h
        ts = jnp.dot(h, ms_ref[...], preferred_element_type=jnp.float32)
        td = jnp.dot(h, md_ref[...], preferred_element_type=jnp.float32)
        ts_ref[...] = ts
        td_ref[...] = td
        m = jnp.concatenate(
            [jnp.max(ts, axis=0)[None, :], jnp.max(td, axis=0)[None, :],
             jnp.zeros((6, 16), jnp.float32)], axis=0)

        @pl.when(i == 0)
        def _():
            mx_ref[...] = m

        @pl.when(i != 0)
        def _():
            mx_ref[...] = jnp.maximum(mx_ref[...], m)

    grid = NP // BLK
    return pl.pallas_call(
        body,
        grid=(grid,),
        in_specs=[
            pl.BlockSpec((BLK, HC), lambda i: (i, 0)),
            pl.BlockSpec((HC, HC), lambda i: (0, 0)),
            pl.BlockSpec((HC, 16), lambda i: (0, 0)),
            pl.BlockSpec((HC, 16), lambda i: (0, 0)),
        ],
        out_specs=[
            pl.BlockSpec((BLK, HC), lambda i: (i, 0)),
            pl.BlockSpec((BLK, 16), lambda i: (i, 0)),
            pl.BlockSpec((BLK, 16), lambda i: (i, 0)),
            pl.BlockSpec((8, 16), lambda i: (0, 0)),
        ],
        out_shape=[
            jax.ShapeDtypeStruct((NP, HC), jnp.float32),
            jax.ShapeDtypeStruct((NP, 16), jnp.float32),
            jax.ShapeDtypeStruct((NP, 16), jnp.float32),
            jax.ShapeDtypeStruct((8, 16), jnp.float32),
        ],
    )(xp, W, Msrc, Mdst)


def _prep_next(part0, part1, bvec, W, Msrc, Mdst, NP, n_valid):
    def body(p0_ref, p1_ref, b_ref, w_ref, ms_ref, md_ref,
             h_ref, ts_ref, td_ref, mx_ref):
        i = pl.program_id(0)
        rows = i * BLK + lax.broadcasted_iota(jnp.int32, (BLK, 1), 0)
        act = jax.nn.relu(p0_ref[...] + p1_ref[...] + b_ref[...])
        act = jnp.where(rows < n_valid, act, 0.0)
        h = jnp.dot(act, w_ref[...], preferred_element_type=jnp.float32)
        h_ref[...] = h
        ts = jnp.dot(h, ms_ref[...], preferred_element_type=jnp.float32)
        td = jnp.dot(h, md_ref[...], preferred_element_type=jnp.float32)
        ts_ref[...] = ts
        td_ref[...] = td
        m = jnp.concatenate(
            [jnp.max(ts, axis=0)[None, :], jnp.max(td, axis=0)[None, :],
             jnp.zeros((6, 16), jnp.float32)], axis=0)

        @pl.when(i == 0)
        def _():
            mx_ref[...] = m

        @pl.when(i != 0)
        def _():
            mx_ref[...] = jnp.maximum(mx_ref[...], m)

    grid = NP // BLK
    return pl.pallas_call(
        body,
        grid=(grid,),
        in_specs=[
            pl.BlockSpec((BLK, HC), lambda i: (i, 0)),
            pl.BlockSpec((BLK, HC), lambda i: (i, 0)),
            pl.BlockSpec((1, HC), lambda i: (0, 0)),
            pl.BlockSpec((HC, HC), lambda i: (0, 0)),
            pl.BlockSpec((HC, 16), lambda i: (0, 0)),
            pl.BlockSpec((HC, 16), lambda i: (0, 0)),
        ],
        out_specs=[
            pl.BlockSpec((BLK, HC), lambda i: (i, 0)),
            pl.BlockSpec((BLK, 16), lambda i: (i, 0)),
            pl.BlockSpec((BLK, 16), lambda i: (i, 0)),
            pl.BlockSpec((8, 16), lambda i: (0, 0)),
        ],
        out_shape=[
            jax.ShapeDtypeStruct((NP, HC), jnp.float32),
            jax.ShapeDtypeStruct((NP, 16), jnp.float32),
            jax.ShapeDtypeStruct((NP, 16), jnp.float32),
            jax.ShapeDtypeStruct((8, 16), jnp.float32),
        ],
    )(part0, part1, bvec, W, Msrc, Mdst)


def _den_combine(den2, NP):
    def body(a_ref, b_ref, o_ref):
        o_ref[...] = 1.0 / (a_ref[0] + b_ref[0] + 1e-16)

    grid = NP // BLK
    return pl.pallas_call(
        body,
        grid=(grid,),
        in_specs=[
            pl.BlockSpec((1, BLK, 16), lambda i: (0, i, 0)),
            pl.BlockSpec((1, BLK, 16), lambda i: (1, i, 0)),
        ],
        out_specs=pl.BlockSpec((BLK, 16), lambda i: (i, 0)),
        out_shape=jax.ShapeDtypeStruct((NP, 16), jnp.float32),
    )(den2, den2)


def _final_proj(part0, part1, bvec, Wp, bp, NP, n_valid):
    def body(p0_ref, p1_ref, b_ref, wp_ref, bp_ref, o_ref):
        i = pl.program_id(0)
        rows = i * BLK + lax.broadcasted_iota(jnp.int32, (BLK, 1), 0)
        act = p0_ref[...] + p1_ref[...] + b_ref[...]
        act = jnp.where(rows < n_valid, act, 0.0)
        o_ref[...] = jnp.dot(act, wp_ref[...],
                             preferred_element_type=jnp.float32) + bp_ref[...]

    grid = NP // BLK
    return pl.pallas_call(
        body,
        grid=(grid,),
        in_specs=[
            pl.BlockSpec((BLK, HC), lambda i: (i, 0)),
            pl.BlockSpec((BLK, HC), lambda i: (i, 0)),
            pl.BlockSpec((1, HC), lambda i: (0, 0)),
            pl.BlockSpec((HC, 3), lambda i: (0, 0)),
            pl.BlockSpec((1, 3), lambda i: (0, 0)),
        ],
        out_specs=pl.BlockSpec((BLK, 3), lambda i: (i, 0)),
        out_shape=jax.ShapeDtypeStruct((n_valid, 3), jnp.float32),
    )(part0, part1, bvec, Wp, bp)


def _sc_edge_logits(tsrc, tdst, b16, srci, dsti, z16, NP, EP):
    per_tile = EP // NW
    nchunks = per_tile // CHUNK
    rows_pt = NP // NS
    mesh = plsc.VectorSubcoreMesh(core_axis_name="c", subcore_axis_name="s",
                                  num_cores=NC, num_subcores=NS)

    @functools.partial(
        pl.kernel, mesh=mesh,
        compiler_params=pltpu.CompilerParams(use_tc_tiling_on_sc=False),
        out_type=[jax.ShapeDtypeStruct((EP, 16), jnp.float32),
                  jax.ShapeDtypeStruct((NC, NP, 16), jnp.float32)],
        scratch_types=[
            pltpu.VMEM((nchunks, CHUNK), jnp.int32),
            pltpu.VMEM((nchunks, CHUNK), jnp.int32),
            pltpu.VMEM((2, CHUNK, 16), jnp.float32),
            pltpu.VMEM((2, CHUNK, 16), jnp.float32),
            pltpu.VMEM((2, CHUNK, 16), jnp.float32),
            pltpu.VMEM((16,), jnp.float32),
            pltpu.VMEM_SHARED((NP, 16), jnp.float32),
            pltpu.SemaphoreType.DMA,
            pltpu.SemaphoreType.DMA,
            pltpu.SemaphoreType.DMA,
        ])
    def k(tsrc_h, tdst_h, b16_h, srci_h, dsti_h, z16_h, p_h, den_h,
          si_v, di_v, s_v, d_v, p_v, b_v, den_sp, gsem, ssem, psem):
        cid = lax.axis_index("c")
        sid = lax.axis_index("s")
        wid = sid * NC + cid
        pltpu.sync_copy(z16_h.at[pl.ds(sid * rows_pt, rows_pt)],
                        den_sp.at[pl.ds(sid * rows_pt, rows_pt)])
        pltpu.sync_copy(b16_h, b_v)
        pltpu.sync_copy(srci_h.at[pl.ds(wid * nchunks, nchunks)], si_v)
        pltpu.sync_copy(dsti_h.at[pl.ds(wid * nchunks, nchunks)], di_v)
        plsc.subcore_barrier()
        bb = b_v[...]
        base_w = wid * per_tile

        def fire(i, buf):
            pltpu.async_copy(tsrc_h.at[si_v.at[i]], s_v.at[buf], gsem)
            pltpu.async_copy(tdst_h.at[di_v.at[i]], d_v.at[buf], gsem)

        fire(0, 0)

        def chunk_body(i, carry):
            cur = lax.rem(i, 2)
            nxt = lax.rem(i + 1, 2)
            base = base_w + i * CHUNK

            @pl.when(i > 0)
            def _():
                pltpu.make_async_copy(
                    p_v.at[nxt], den_sp.at[di_v.at[i - 1]], ssem).wait()
                pltpu.make_async_copy(
                    p_v.at[nxt], p_h.at[pl.ds(base - CHUNK, CHUNK)],
                    psem).wait()

            @pl.when(i + 1 < nchunks)
            def _():
                fire(i + 1, nxt)

            pltpu.make_async_copy(tsrc_h.at[si_v.at[i]], s_v.at[cur],
                                  gsem).wait()
            pltpu.make_async_copy(tdst_h.at[di_v.at[i]], d_v.at[cur],
                                  gsem).wait()

            @plsc.parallel_loop(0, CHUNK, unroll=8)
            def edge_body(e):
                ev = s_v[cur, e] + d_v[cur, e]
                ev = jnp.maximum(ev, 0.0) + 0.2 * jnp.minimum(ev, 0.0)
                p_v[cur, e] = jnp.exp(ev - bb)
            pltpu.async_copy(p_v.at[cur], den_sp.at[di_v.at[i]], ssem,
                             add=True)
            pltpu.async_copy(p_v.at[cur], p_h.at[pl.ds(base, CHUNK)], psem)
            return carry

        lax.fori_loop(0, nchunks, chunk_body, 0)
        last = lax.rem(nchunks - 1, 2)
        pltpu.make_async_copy(
            p_v.at[last], den_sp.at[di_v.at[nchunks - 1]], ssem).wait()
        pltpu.make_async_copy(
            p_v.at[last],
            p_h.at[pl.ds(base_w + (nchunks - 1) * CHUNK, CHUNK)], psem).wait()
        plsc.subcore_barrier()
        pltpu.sync_copy(den_sp.at[pl.ds(sid * rows_pt, rows_pt)],
                        den_h.at[cid, pl.ds(sid * rows_pt, rows_pt)])

    return k(tsrc, tdst, b16, srci, dsti, z16)


def _sc_messages(h, den0, den1, p, srci, dsti, z128, NP, EP, n):
    per_tile = EP // NW
    nchunks = per_tile // CHUNK
    NPS = _round_up(n + 1, NS)
    rows_pt = NPS // NS
    mesh = plsc.VectorSubcoreMesh(core_axis_name="c", subcore_axis_name="s",
                                  num_cores=NC, num_subcores=NS)

    @functools.partial(
        pl.kernel, mesh=mesh,
        compiler_params=pltpu.CompilerParams(use_tc_tiling_on_sc=False),
        out_type=jax.ShapeDtypeStruct((NC, NP, HC), jnp.float32),
        scratch_types=[
            pltpu.VMEM((3, CHUNK), jnp.int32),
            pltpu.VMEM((3, CHUNK), jnp.int32),
            pltpu.VMEM((2, CHUNK, 16), jnp.float32),
            pltpu.VMEM((2, CHUNK, 16), jnp.float32),
            pltpu.VMEM((2, CHUNK, 16), jnp.float32),
            pltpu.VMEM((2, CHUNK, HC), jnp.float32),
            pltpu.VMEM_SHARED((NPS, HC), jnp.float32),
            pltpu.SemaphoreType.DMA,
            pltpu.SemaphoreType.DMA,
            pltpu.SemaphoreType.DMA,
        ])
    def k(h_h, d0_h, d1_h, p_h, srci_h, dsti_h, z128_h, out_h,
          si_v, di_v, pa_v, d0_v, d1_v, hs_v, out_sp, gsem, ssem, isem):
        cid = lax.axis_index("c")
        sid = lax.axis_index("s")
        wid = sid * NC + cid
        pltpu.sync_copy(z128_h.at[pl.ds(sid * rows_pt, rows_pt)],
                        out_sp.at[pl.ds(sid * rows_pt, rows_pt)])
        plsc.subcore_barrier()
        base_w = wid * nchunks

        def load_idx(i, slot):
            pltpu.async_copy(srci_h.at[pl.ds(base_w + i, 1)],
                             si_v.at[pl.ds(slot, 1)], isem)
            pltpu.async_copy(dsti_h.at[pl.ds(base_w + i, 1)],
                             di_v.at[pl.ds(slot, 1)], isem)

        def fire(i, buf, slot):
            pltpu.make_async_copy(srci_h.at[pl.ds(base_w + i, 1)],
                                  si_v.at[pl.ds(slot, 1)], isem).wait()
            pltpu.make_async_copy(dsti_h.at[pl.ds(base_w + i, 1)],
                                  di_v.at[pl.ds(slot, 1)], isem).wait()
            base = (base_w + i) * CHUNK
            pltpu.async_copy(d0_h.at[di_v.at[slot]], d0_v.at[buf], gsem)
            pltpu.async_copy(d1_h.at[di_v.at[slot]], d1_v.at[buf], gsem)
            pltpu.async_copy(p_h.at[pl.ds(base, CHUNK)], pa_v.at[buf], gsem)
            pltpu.async_copy(h_h.at[si_v.at[slot]], hs_v.at[buf], gsem)

        load_idx(0, 0)
        fire(0, 0, 0)
        load_idx(1, 1)

        def chunk_body(i, carry):
            cur = lax.rem(i, 2)
            nxt = lax.rem(i + 1, 2)
            slot = lax.rem(i, 3)
            slot1 = lax.rem(i + 1, 3)
            slot2 = lax.rem(i + 2, 3)
            base = (base_w + i) * CHUNK

            @pl.when(i > 0)
            def _():
                pltpu.make_async_copy(
                    hs_v.at[nxt], out_sp.at[di_v.at[slot2]], ssem).wait()

            @pl.when(i + 1 < nchunks)
            def _():
                fire(i + 1, nxt, slot1)

            @pl.when(i + 2 < nchunks)
            def _():
                load_idx(i + 2, slot2)

            pltpu.make_async_copy(d0_h.at[di_v.at[slot]], d0_v.at[cur],
                                  gsem).wait()
            pltpu.make_async_copy(d1_h.at[di_v.at[slot]], d1_v.at[cur],
                                  gsem).wait()
            pltpu.make_async_copy(p_h.at[pl.ds(base, CHUNK)], pa_v.at[cur],
                                  gsem).wait()
            pltpu.make_async_copy(h_h.at[si_v.at[slot]], hs_v.at[cur],
                                  gsem).wait()

            @plsc.parallel_loop(0, CHUNK, unroll=4)
            def edge_body(e):
                al = pa_v[cur, e] / (d0_v[cur, e] + d1_v[cur, e] + 1e-16)
                for j in range(H):
                    sc = al[j]
                    hs_v[cur, e, pl.ds(j * 16, 16)] = (
                        hs_v[cur, e, pl.ds(j * 16, 16)] * sc)
            pltpu.async_copy(hs_v.at[cur], out_sp.at[di_v.at[slot]], ssem,
                             add=True)
            return carry

        lax.fori_loop(0, nchunks, chunk_body, 0)
        last = lax.rem(nchunks - 1, 2)
        pltpu.make_async_copy(
            hs_v.at[last],
            out_sp.at[di_v.at[lax.rem(nchunks - 1, 3)]], ssem).wait()
        plsc.subcore_barrier()
        pltpu.sync_copy(out_sp.at[pl.ds(sid * rows_pt, rows_pt)],
                        out_h.at[cid, pl.ds(sid * rows_pt, rows_pt)])

    return k(h, den0, den1, p, srci, dsti, z128)


def _mmat(a):
    af = a.reshape(HC)
    ci = jnp.arange(HC, dtype=jnp.int32) // C
    jj = jnp.arange(16, dtype=jnp.int32) % H
    mask = (ci[:, None] == jj[None, :]).astype(jnp.float32)
    return af[:, None] * mask


def kernel(x, edge_index, batch, W0, a_src0, a_dst0, b0, W1, a_src1, a_dst1,
           b1, W2, a_src2, a_dst2, b2, Wp, bp):
    n = x.shape[0]
    e = edge_index.shape[1]
    NP = _round_up(n + 1, BLK)
    ep_raw = e + n
    per_tile = _round_up(-(-ep_raw // NW), CHUNK)
    EP = per_tile * NW

    loop = jnp.arange(n, dtype=jnp.int32)
    srci = jnp.concatenate(
        [edge_index[0], loop,
         jnp.zeros((EP - ep_raw,), jnp.int32)]).reshape(EP // CHUNK, CHUNK)
    dsti = jnp.concatenate(
        [edge_index[1], loop,
         jnp.full((EP - ep_raw,), n, jnp.int32)]).reshape(EP // CHUNK, CHUNK)

    xp = jnp.pad(x, ((0, NP - n), (0, 0)))
    z16 = jnp.zeros((NP, 16), jnp.float32)
    z128 = jnp.zeros((NP, HC), jnp.float32)

    Ws = [W0, W1, W2]
    Ms = [(_mmat(a_src0), _mmat(a_dst0)),
          (_mmat(a_src1), _mmat(a_dst1)),
          (_mmat(a_src2), _mmat(a_dst2))]
    bs = [b0.reshape(1, HC), b1.reshape(1, HC), b2.reshape(1, HC)]

    part0 = part1 = None
    for l in range(3):
        if l == 0:
            hmat, ts, td, mx = _prep_first(xp, Ws[0], Ms[0][0], Ms[0][1], NP)
        else:
            hmat, ts, td, mx = _prep_next(part0, part1, bs[l - 1], Ws[l],
                                          Ms[l][0], Ms[l][1], NP, n)
        b16 = mx[0] + mx[1]
        p, den2 = _sc_edge_logits(ts, td, b16, srci, dsti, z16, NP, EP)
        out2 = _sc_messages(hmat, den2[0], den2[1], p, srci, dsti, z128,
                            NP, EP, n)
        part0, part1 = out2[0], out2[1]

    return _final_proj(part0, part1, bs[2], Wp, bp.reshape(1, 3), NP, n)


# speedup vs baseline: 110.5385x; 1.0287x over previous
import functools

import jax
import jax.numpy as jnp
from jax import lax
from jax.experimental import pallas as pl
from jax.experimental.pallas import tpu as pltpu
from jax.experimental.pallas import tpu_sc as plsc

NC, NS = 2, 16
NW = NC * NS
H, C, HC = 8, 16, 128
CHUNK = 128
BLK = 512
SC0_PCT = 58


def _core_split(n_pair):
    n0 = (n_pair * SC0_PCT + 50) // 100
    return n0, n_pair - n0


def _round_up(a, b):
    return (a + b - 1) // b * b


def _prep_first(xp, W, Msrc, Mdst, NP):
    def body(x_ref, w_ref, ms_ref, md_ref, h_ref, ts_ref, td_ref, mx_ref):
        i = pl.program_id(0)
        h = jnp.dot(x_ref[...], w_ref[...], preferred_element_type=jnp.float32)
        h_ref[...] = h
        ts = jnp.dot(h, ms_ref[...], preferred_element_type=jnp.float32)
        td = jnp.dot(h, md_ref[...], preferred_element_type=jnp.float32)
        ts_ref[...] = ts
        td_ref[...] = td
        m = jnp.concatenate(
            [jnp.max(ts, axis=0)[None, :], jnp.max(td, axis=0)[None, :],
             jnp.zeros((6, 16), jnp.float32)], axis=0)

        @pl.when(i == 0)
        def _():
            mx_ref[...] = m

        @pl.when(i != 0)
        def _():
            mx_ref[...] = jnp.maximum(mx_ref[...], m)

    grid = NP // BLK
    return pl.pallas_call(
        body,
        grid=(grid,),
        in_specs=[
            pl.BlockSpec((BLK, HC), lambda i: (i, 0)),
            pl.BlockSpec((HC, HC), lambda i: (0, 0)),
            pl.BlockSpec((HC, 16), lambda i: (0, 0)),
            pl.BlockSpec((HC, 16), lambda i: (0, 0)),
        ],
        out_specs=[
            pl.BlockSpec((BLK, HC), lambda i: (i, 0)),
            pl.BlockSpec((BLK, 16), lambda i: (i, 0)),
            pl.BlockSpec((BLK, 16), lambda i: (i, 0)),
            pl.BlockSpec((8, 16), lambda i: (0, 0)),
        ],
        out_shape=[
            jax.ShapeDtypeStruct((NP, HC), jnp.float32),
            jax.ShapeDtypeStruct((NP, 16), jnp.float32),
            jax.ShapeDtypeStruct((NP, 16), jnp.float32),
            jax.ShapeDtypeStruct((8, 16), jnp.float32),
        ],
    )(xp, W, Msrc, Mdst)


def _prep_next(part0, part1, bvec, W, Msrc, Mdst, NP, n_valid):
    def body(p0_ref, p1_ref, b_ref, w_ref, ms_ref, md_ref,
             h_ref, ts_ref, td_ref, mx_ref):
        i = pl.program_id(0)
        rows = i * BLK + lax.broadcasted_iota(jnp.int32, (BLK, 1), 0)
        act = jax.nn.relu(p0_ref[...] + p1_ref[...] + b_ref[...])
        act = jnp.where(rows < n_valid, act, 0.0)
        h = jnp.dot(act, w_ref[...], preferred_element_type=jnp.float32)
        h_ref[...] = h
        ts = jnp.dot(h, ms_ref[...], preferred_element_type=jnp.float32)
        td = jnp.dot(h, md_ref[...], preferred_element_type=jnp.float32)
        ts_ref[...] = ts
        td_ref[...] = td
        m = jnp.concatenate(
            [jnp.max(ts, axis=0)[None, :], jnp.max(td, axis=0)[None, :],
             jnp.zeros((6, 16), jnp.float32)], axis=0)

        @pl.when(i == 0)
        def _():
            mx_ref[...] = m

        @pl.when(i != 0)
        def _():
            mx_ref[...] = jnp.maximum(mx_ref[...], m)

    grid = NP // BLK
    return pl.pallas_call(
        body,
        grid=(grid,),
        in_specs=[
            pl.BlockSpec((BLK, HC), lambda i: (i, 0)),
            pl.BlockSpec((BLK, HC), lambda i: (i, 0)),
            pl.BlockSpec((1, HC), lambda i: (0, 0)),
            pl.BlockSpec((HC, HC), lambda i: (0, 0)),
            pl.BlockSpec((HC, 16), lambda i: (0, 0)),
            pl.BlockSpec((HC, 16), lambda i: (0, 0)),
        ],
        out_specs=[
            pl.BlockSpec((BLK, HC), lambda i: (i, 0)),
            pl.BlockSpec((BLK, 16), lambda i: (i, 0)),
            pl.BlockSpec((BLK, 16), lambda i: (i, 0)),
            pl.BlockSpec((8, 16), lambda i: (0, 0)),
        ],
        out_shape=[
            jax.ShapeDtypeStruct((NP, HC), jnp.float32),
            jax.ShapeDtypeStruct((NP, 16), jnp.float32),
            jax.ShapeDtypeStruct((NP, 16), jnp.float32),
            jax.ShapeDtypeStruct((8, 16), jnp.float32),
        ],
    )(part0, part1, bvec, W, Msrc, Mdst)


def _den_combine(den2, NP):
    def body(a_ref, b_ref, o_ref):
        o_ref[...] = 1.0 / (a_ref[0] + b_ref[0] + 1e-16)

    grid = NP // BLK
    return pl.pallas_call(
        body,
        grid=(grid,),
        in_specs=[
            pl.BlockSpec((1, BLK, 16), lambda i: (0, i, 0)),
            pl.BlockSpec((1, BLK, 16), lambda i: (1, i, 0)),
        ],
        out_specs=pl.BlockSpec((BLK, 16), lambda i: (i, 0)),
        out_shape=jax.ShapeDtypeStruct((NP, 16), jnp.float32),
    )(den2, den2)


def _final_proj(part0, part1, bvec, Wp, bp, NP, n_valid):
    def body(p0_ref, p1_ref, b_ref, wp_ref, bp_ref, o_ref):
        i = pl.program_id(0)
        rows = i * BLK + lax.broadcasted_iota(jnp.int32, (BLK, 1), 0)
        act = p0_ref[...] + p1_ref[...] + b_ref[...]
        act = jnp.where(rows < n_valid, act, 0.0)
        o_ref[...] = jnp.dot(act, wp_ref[...],
                             preferred_element_type=jnp.float32) + bp_ref[...]

    grid = NP // BLK
    return pl.pallas_call(
        body,
        grid=(grid,),
        in_specs=[
            pl.BlockSpec((BLK, HC), lambda i: (i, 0)),
            pl.BlockSpec((BLK, HC), lambda i: (i, 0)),
            pl.BlockSpec((1, HC), lambda i: (0, 0)),
            pl.BlockSpec((HC, 3), lambda i: (0, 0)),
            pl.BlockSpec((1, 3), lambda i: (0, 0)),
        ],
        out_specs=pl.BlockSpec((BLK, 3), lambda i: (i, 0)),
        out_shape=jax.ShapeDtypeStruct((n_valid, 3), jnp.float32),
    )(part0, part1, bvec, Wp, bp)


def _sc_edge_logits(tsrc, tdst, b16, srci, dsti, z16, NP, EP):
    n_pair = (EP // CHUNK) // NS
    N0, N1 = _core_split(n_pair)
    rows_pt = NP // NS
    mesh = plsc.VectorSubcoreMesh(core_axis_name="c", subcore_axis_name="s",
                                  num_cores=NC, num_subcores=NS)

    @functools.partial(
        pl.kernel, mesh=mesh,
        compiler_params=pltpu.CompilerParams(use_tc_tiling_on_sc=False),
        out_type=[jax.ShapeDtypeStruct((EP, 16), jnp.float32),
                  jax.ShapeDtypeStruct((NC, NP, 16), jnp.float32)],
        scratch_types=[
            pltpu.VMEM((N0, CHUNK), jnp.int32),
            pltpu.VMEM((N0, CHUNK), jnp.int32),
            pltpu.VMEM((2, CHUNK, 16), jnp.float32),
            pltpu.VMEM((2, CHUNK, 16), jnp.float32),
            pltpu.VMEM((2, CHUNK, 16), jnp.float32),
            pltpu.VMEM((16,), jnp.float32),
            pltpu.VMEM_SHARED((NP, 16), jnp.float32),
            pltpu.SemaphoreType.DMA,
            pltpu.SemaphoreType.DMA,
            pltpu.SemaphoreType.DMA,
        ])
    def k(tsrc_h, tdst_h, b16_h, srci_h, dsti_h, z16_h, p_h, den_h,
          si_v, di_v, s_v, d_v, p_v, b_v, den_sp, gsem, ssem, psem):
        cid = lax.axis_index("c")
        sid = lax.axis_index("s")
        n_my = jnp.where(cid == 0, N0, N1)
        base_c = jnp.where(cid == 0, sid * N0, NS * N0 + sid * N1)
        pltpu.sync_copy(z16_h.at[pl.ds(sid * rows_pt, rows_pt)],
                        den_sp.at[pl.ds(sid * rows_pt, rows_pt)])
        pltpu.sync_copy(b16_h, b_v)
        pltpu.sync_copy(srci_h.at[pl.ds(base_c, N0)], si_v)
        pltpu.sync_copy(dsti_h.at[pl.ds(base_c, N0)], di_v)
        plsc.subcore_barrier()
        bb = b_v[...]

        def fire(i, buf):
            pltpu.async_copy(tsrc_h.at[si_v.at[i]], s_v.at[buf], gsem)
            pltpu.async_copy(tdst_h.at[di_v.at[i]], d_v.at[buf], gsem)

        fire(0, 0)

        def chunk_body(i, carry):
            cur = lax.rem(i, 2)
            nxt = lax.rem(i + 1, 2)
            base = (base_c + i) * CHUNK

            @pl.when(i > 0)
            def _():
                pltpu.make_async_copy(
                    p_v.at[nxt], den_sp.at[di_v.at[i - 1]], ssem).wait()
                pltpu.make_async_copy(
                    p_v.at[nxt], p_h.at[pl.ds(base - CHUNK, CHUNK)],
                    psem).wait()

            @pl.when(i + 1 < n_my)
            def _():
                fire(i + 1, nxt)

            pltpu.make_async_copy(tsrc_h.at[si_v.at[i]], s_v.at[cur],
                                  gsem).wait()
            pltpu.make_async_copy(tdst_h.at[di_v.at[i]], d_v.at[cur],
                                  gsem).wait()

            @plsc.parallel_loop(0, CHUNK, unroll=8)
            def edge_body(e):
                ev = s_v[cur, e] + d_v[cur, e]
                ev = jnp.maximum(ev, 0.0) + 0.2 * jnp.minimum(ev, 0.0)
                p_v[cur, e] = jnp.exp(ev - bb)
            pltpu.async_copy(p_v.at[cur], den_sp.at[di_v.at[i]], ssem,
                             add=True)
            pltpu.async_copy(p_v.at[cur], p_h.at[pl.ds(base, CHUNK)], psem)
            return carry

        lax.fori_loop(0, n_my, chunk_body, 0)
        last = lax.rem(n_my - 1, 2)
        pltpu.make_async_copy(
            p_v.at[last], den_sp.at[di_v.at[n_my - 1]], ssem).wait()
        pltpu.make_async_copy(
            p_v.at[last],
            p_h.at[pl.ds((base_c + n_my - 1) * CHUNK, CHUNK)], psem).wait()
        plsc.subcore_barrier()
        pltpu.sync_copy(den_sp.at[pl.ds(sid * rows_pt, rows_pt)],
                        den_h.at[cid, pl.ds(sid * rows_pt, rows_pt)])

    return k(tsrc, tdst, b16, srci, dsti, z16)


def _sc_messages(h, den0, den1, p, srci, dsti, z128, NP, EP, n):
    n_pair = (EP // CHUNK) // NS
    N0, N1 = _core_split(n_pair)
    NPS = _round_up(n + 1, NS)
    rows_pt = NPS // NS
    mesh = plsc.VectorSubcoreMesh(core_axis_name="c", subcore_axis_name="s",
                                  num_cores=NC, num_subcores=NS)

    @functools.partial(
        pl.kernel, mesh=mesh,
        compiler_params=pltpu.CompilerParams(use_tc_tiling_on_sc=False),
        out_type=jax.ShapeDtypeStruct((NC, NP, HC), jnp.float32),
        scratch_types=[
            pltpu.VMEM((3, CHUNK), jnp.int32),
            pltpu.VMEM((3, CHUNK), jnp.int32),
            pltpu.VMEM((2, CHUNK, 16), jnp.float32),
            pltpu.VMEM((2, CHUNK, 16), jnp.float32),
            pltpu.VMEM((2, CHUNK, 16), jnp.float32),
            pltpu.VMEM((2, CHUNK, HC), jnp.float32),
            pltpu.VMEM_SHARED((NPS, HC), jnp.float32),
            pltpu.SemaphoreType.DMA,
            pltpu.SemaphoreType.DMA,
            pltpu.SemaphoreType.DMA,
        ])
    def k(h_h, d0_h, d1_h, p_h, srci_h, dsti_h, z128_h, out_h,
          si_v, di_v, pa_v, d0_v, d1_v, hs_v, out_sp, gsem, ssem, isem):
        cid = lax.axis_index("c")
        sid = lax.axis_index("s")
        n_my = jnp.where(cid == 0, N0, N1)
        base_w = jnp.where(cid == 0, sid * N0, NS * N0 + sid * N1)
        pltpu.sync_copy(z128_h.at[pl.ds(sid * rows_pt, rows_pt)],
                        out_sp.at[pl.ds(sid * rows_pt, rows_pt)])
        plsc.subcore_barrier()

        def load_idx(i, slot):
            pltpu.async_copy(srci_h.at[pl.ds(base_w + i, 1)],
                             si_v.at[pl.ds(slot, 1)], isem)
            pltpu.async_copy(dsti_h.at[pl.ds(base_w + i, 1)],
                             di_v.at[pl.ds(slot, 1)], isem)

        def fire(i, buf, slot):
            pltpu.make_async_copy(srci_h.at[pl.ds(base_w + i, 1)],
                                  si_v.at[pl.ds(slot, 1)], isem).wait()
            pltpu.make_async_copy(dsti_h.at[pl.ds(base_w + i, 1)],
                                  di_v.at[pl.ds(slot, 1)], isem).wait()
            base = (base_w + i) * CHUNK
            pltpu.async_copy(d0_h.at[di_v.at[slot]], d0_v.at[buf], gsem)
            pltpu.async_copy(d1_h.at[di_v.at[slot]], d1_v.at[buf], gsem)
            pltpu.async_copy(p_h.at[pl.ds(base, CHUNK)], pa_v.at[buf], gsem)
            pltpu.async_copy(h_h.at[si_v.at[slot]], hs_v.at[buf], gsem)

        load_idx(0, 0)
        fire(0, 0, 0)
        load_idx(1, 1)

        def chunk_body(i, carry):
            cur = lax.rem(i, 2)
            nxt = lax.rem(i + 1, 2)
            slot = lax.rem(i, 3)
            slot1 = lax.rem(i + 1, 3)
            slot2 = lax.rem(i + 2, 3)
            base = (base_w + i) * CHUNK

            @pl.when(i > 0)
            def _():
                pltpu.make_async_copy(
                    hs_v.at[nxt], out_sp.at[di_v.at[slot2]], ssem).wait()

            @pl.when(i + 1 < n_my)
            def _():
                fire(i + 1, nxt, slot1)

            @pl.when(i + 2 < n_my)
            def _():
                load_idx(i + 2, slot2)

            pltpu.make_async_copy(d0_h.at[di_v.at[slot]], d0_v.at[cur],
                                  gsem).wait()
            pltpu.make_async_copy(d1_h.at[di_v.at[slot]], d1_v.at[cur],
                                  gsem).wait()
            pltpu.make_async_copy(p_h.at[pl.ds(base, CHUNK)], pa_v.at[cur],
                                  gsem).wait()
            pltpu.make_async_copy(h_h.at[si_v.at[slot]], hs_v.at[cur],
                                  gsem).wait()

            @plsc.parallel_loop(0, CHUNK, unroll=4)
            def edge_body(e):
                al = pa_v[cur, e] / (d0_v[cur, e] + d1_v[cur, e] + 1e-16)
                for j in range(H):
                    sc = al[j]
                    hs_v[cur, e, pl.ds(j * 16, 16)] = (
                        hs_v[cur, e, pl.ds(j * 16, 16)] * sc)
            pltpu.async_copy(hs_v.at[cur], out_sp.at[di_v.at[slot]], ssem,
                             add=True)
            return carry

        lax.fori_loop(0, n_my, chunk_body, 0)
        last = lax.rem(n_my - 1, 2)
        pltpu.make_async_copy(
            hs_v.at[last],
            out_sp.at[di_v.at[lax.rem(n_my - 1, 3)]], ssem).wait()
        plsc.subcore_barrier()
        pltpu.sync_copy(out_sp.at[pl.ds(sid * rows_pt, rows_pt)],
                        out_h.at[cid, pl.ds(sid * rows_pt, rows_pt)])

    return k(h, den0, den1, p, srci, dsti, z128)


def _mmat(a):
    af = a.reshape(HC)
    ci = jnp.arange(HC, dtype=jnp.int32) // C
    jj = jnp.arange(16, dtype=jnp.int32) % H
    mask = (ci[:, None] == jj[None, :]).astype(jnp.float32)
    return af[:, None] * mask


def kernel(x, edge_index, batch, W0, a_src0, a_dst0, b0, W1, a_src1, a_dst1,
           b1, W2, a_src2, a_dst2, b2, Wp, bp):
    n = x.shape[0]
    e = edge_index.shape[1]
    NP = _round_up(n + 1, BLK)
    ep_raw = e + n
    per_tile = _round_up(-(-ep_raw // NW), CHUNK)
    EP = per_tile * NW

    loop = jnp.arange(n, dtype=jnp.int32)
    srci = jnp.pad(jnp.concatenate(
        [edge_index[0], loop,
         jnp.zeros((EP - ep_raw,), jnp.int32)]).reshape(EP // CHUNK, CHUNK),
        ((0, 32), (0, 0)))
    dsti = jnp.pad(jnp.concatenate(
        [edge_index[1], loop,
         jnp.full((EP - ep_raw,), n, jnp.int32)]).reshape(EP // CHUNK, CHUNK),
        ((0, 32), (0, 0)))

    xp = jnp.pad(x, ((0, NP - n), (0, 0)))
    z16 = jnp.zeros((NP, 16), jnp.float32)
    z128 = jnp.zeros((NP, HC), jnp.float32)

    Ws = [W0, W1, W2]
    Ms = [(_mmat(a_src0), _mmat(a_dst0)),
          (_mmat(a_src1), _mmat(a_dst1)),
          (_mmat(a_src2), _mmat(a_dst2))]
    bs = [b0.reshape(1, HC), b1.reshape(1, HC), b2.reshape(1, HC)]

    part0 = part1 = None
    for l in range(3):
        if l == 0:
            hmat, ts, td, mx = _prep_first(xp, Ws[0], Ms[0][0], Ms[0][1], NP)
        else:
            hmat, ts, td, mx = _prep_next(part0, part1, bs[l - 1], Ws[l],
                                          Ms[l][0], Ms[l][1], NP, n)
        b16 = mx[0] + mx[1]
        p, den2 = _sc_edge_logits(ts, td, b16, srci, dsti, z16, NP, EP)
        out2 = _sc_messages(hmat, den2[0], den2[1], p, srci, dsti, z128,
                            NP, EP, n)
        part0, part1 = out2[0], out2[1]

    return _final_proj(part0, part1, bs[2], Wp, bp.reshape(1, 3), NP, n)


# speedup vs baseline: 110.7370x; 1.0018x over previous
import functools

import jax
import jax.numpy as jnp
from jax import lax
from jax.experimental import pallas as pl
from jax.experimental.pallas import tpu as pltpu
from jax.experimental.pallas import tpu_sc as plsc

NC, NS = 2, 16
NW = NC * NS
H, C, HC = 8, 16, 128
CHUNK = 128
BLK = 512
SC0_PCT = 56


def _core_split(n_pair):
    n0 = (n_pair * SC0_PCT + 50) // 100
    return n0, n_pair - n0


def _round_up(a, b):
    return (a + b - 1) // b * b


def _prep_first(xp, W, Msrc, Mdst, NP):
    def body(x_ref, w_ref, ms_ref, md_ref, h_ref, ts_ref, td_ref, mx_ref):
        i = pl.program_id(0)
        h = jnp.dot(x_ref[...], w_ref[...], preferred_element_type=jnp.float32)
        h_ref[...] = h
        ts = jnp.dot(h, ms_ref[...], preferred_element_type=jnp.float32)
        td = jnp.dot(h, md_ref[...], preferred_element_type=jnp.float32)
        ts_ref[...] = ts
        td_ref[...] = td
        m = jnp.concatenate(
            [jnp.max(ts, axis=0)[None, :], jnp.max(td, axis=0)[None, :],
             jnp.zeros((6, 16), jnp.float32)], axis=0)

        @pl.when(i == 0)
        def _():
            mx_ref[...] = m

        @pl.when(i != 0)
        def _():
            mx_ref[...] = jnp.maximum(mx_ref[...], m)

    grid = NP // BLK
    return pl.pallas_call(
        body,
        grid=(grid,),
        in_specs=[
            pl.BlockSpec((BLK, HC), lambda i: (i, 0)),
            pl.BlockSpec((HC, HC), lambda i: (0, 0)),
            pl.BlockSpec((HC, 16), lambda i: (0, 0)),
            pl.BlockSpec((HC, 16), lambda i: (0, 0)),
        ],
        out_specs=[
            pl.BlockSpec((BLK, HC), lambda i: (i, 0)),
            pl.BlockSpec((BLK, 16), lambda i: (i, 0)),
            pl.BlockSpec((BLK, 16), lambda i: (i, 0)),
            pl.BlockSpec((8, 16), lambda i: (0, 0)),
        ],
        out_shape=[
            jax.ShapeDtypeStruct((NP, HC), jnp.float32),
            jax.ShapeDtypeStruct((NP, 16), jnp.float32),
            jax.ShapeDtypeStruct((NP, 16), jnp.float32),
            jax.ShapeDtypeStruct((8, 16), jnp.float32),
        ],
    )(xp, W, Msrc, Mdst)


def _prep_next(part0, part1, bvec, W, Msrc, Mdst, NP, n_valid):
    def body(p0_ref, p1_ref, b_ref, w_ref, ms_ref, md_ref,
             h_ref, ts_ref, td_ref, mx_ref):
        i = pl.program_id(0)
        rows = i * BLK + lax.broadcasted_iota(jnp.int32, (BLK, 1), 0)
        act = jax.nn.relu(p0_ref[...] + p1_ref[...] + b_ref[...])
        act = jnp.where(rows < n_valid, act, 0.0)
        h = jnp.dot(act, w_ref[...], preferred_element_type=jnp.float32)
        h_ref[...] = h
        ts = jnp.dot(h, ms_ref[...], preferred_element_type=jnp.float32)
        td = jnp.dot(h, md_ref[...], preferred_element_type=jnp.float32)
        ts_ref[...] = ts
        td_ref[...] = td
        m = jnp.concatenate(
            [jnp.max(ts, axis=0)[None, :], jnp.max(td, axis=0)[None, :],
             jnp.zeros((6, 16), jnp.float32)], axis=0)

        @pl.when(i == 0)
        def _():
            mx_ref[...] = m

        @pl.when(i != 0)
        def _():
            mx_ref[...] = jnp.maximum(mx_ref[...], m)

    grid = NP // BLK
    return pl.pallas_call(
        body,
        grid=(grid,),
        in_specs=[
            pl.BlockSpec((BLK, HC), lambda i: (i, 0)),
            pl.BlockSpec((BLK, HC), lambda i: (i, 0)),
            pl.BlockSpec((1, HC), lambda i: (0, 0)),
            pl.BlockSpec((HC, HC), lambda i: (0, 0)),
            pl.BlockSpec((HC, 16), lambda i: (0, 0)),
            pl.BlockSpec((HC, 16), lambda i: (0, 0)),
        ],
        out_specs=[
            pl.BlockSpec((BLK, HC), lambda i: (i, 0)),
            pl.BlockSpec((BLK, 16), lambda i: (i, 0)),
            pl.BlockSpec((BLK, 16), lambda i: (i, 0)),
            pl.BlockSpec((8, 16), lambda i: (0, 0)),
        ],
        out_shape=[
            jax.ShapeDtypeStruct((NP, HC), jnp.float32),
            jax.ShapeDtypeStruct((NP, 16), jnp.float32),
            jax.ShapeDtypeStruct((NP, 16), jnp.float32),
            jax.ShapeDtypeStruct((8, 16), jnp.float32),
        ],
    )(part0, part1, bvec, W, Msrc, Mdst)


def _den_combine(den2, NP):
    def body(a_ref, b_ref, o_ref):
        o_ref[...] = 1.0 / (a_ref[0] + b_ref[0] + 1e-16)

    grid = NP // BLK
    return pl.pallas_call(
        body,
        grid=(grid,),
        in_specs=[
            pl.BlockSpec((1, BLK, 16), lambda i: (0, i, 0)),
            pl.BlockSpec((1, BLK, 16), lambda i: (1, i, 0)),
        ],
        out_specs=pl.BlockSpec((BLK, 16), lambda i: (i, 0)),
        out_shape=jax.ShapeDtypeStruct((NP, 16), jnp.float32),
    )(den2, den2)


def _final_proj(part0, part1, bvec, Wp, bp, NP, n_valid):
    def body(p0_ref, p1_ref, b_ref, wp_ref, bp_ref, o_ref):
        i = pl.program_id(0)
        rows = i * BLK + lax.broadcasted_iota(jnp.int32, (BLK, 1), 0)
        act = p0_ref[...] + p1_ref[...] + b_ref[...]
        act = jnp.where(rows < n_valid, act, 0.0)
        o_ref[...] = jnp.dot(act, wp_ref[...],
                             preferred_element_type=jnp.float32) + bp_ref[...]

    grid = NP // BLK
    return pl.pallas_call(
        body,
        grid=(grid,),
        in_specs=[
            pl.BlockSpec((BLK, HC), lambda i: (i, 0)),
            pl.BlockSpec((BLK, HC), lambda i: (i, 0)),
            pl.BlockSpec((1, HC), lambda i: (0, 0)),
            pl.BlockSpec((HC, 3), lambda i: (0, 0)),
            pl.BlockSpec((1, 3), lambda i: (0, 0)),
        ],
        out_specs=pl.BlockSpec((BLK, 3), lambda i: (i, 0)),
        out_shape=jax.ShapeDtypeStruct((n_valid, 3), jnp.float32),
    )(part0, part1, bvec, Wp, bp)


def _sc_edge_logits(tsrc, tdst, b16, srci, dsti, NP, EP):
    n_pair = (EP // CHUNK) // NS
    N0, N1 = _core_split(n_pair)
    rows_pt = NP // NS
    mesh = plsc.VectorSubcoreMesh(core_axis_name="c", subcore_axis_name="s",
                                  num_cores=NC, num_subcores=NS)

    @functools.partial(
        pl.kernel, mesh=mesh,
        compiler_params=pltpu.CompilerParams(use_tc_tiling_on_sc=False),
        out_type=[jax.ShapeDtypeStruct((EP, 16), jnp.float32),
                  jax.ShapeDtypeStruct((NC, NP, 16), jnp.float32)],
        scratch_types=[
            pltpu.VMEM((N0, CHUNK), jnp.int32),
            pltpu.VMEM((N0, CHUNK), jnp.int32),
            pltpu.VMEM((2, CHUNK, 16), jnp.float32),
            pltpu.VMEM((2, CHUNK, 16), jnp.float32),
            pltpu.VMEM((2, CHUNK, 16), jnp.float32),
            pltpu.VMEM((16,), jnp.float32),
            pltpu.VMEM_SHARED((NP, 16), jnp.float32),
            pltpu.SemaphoreType.DMA,
            pltpu.SemaphoreType.DMA,
            pltpu.SemaphoreType.DMA,
        ])
    def k(tsrc_h, tdst_h, b16_h, srci_h, dsti_h, p_h, den_h,
          si_v, di_v, s_v, d_v, p_v, b_v, den_sp, gsem, ssem, psem):
        cid = lax.axis_index("c")
        sid = lax.axis_index("s")
        n_my = jnp.where(cid == 0, N0, N1)
        base_c = jnp.where(cid == 0, sid * N0, NS * N0 + sid * N1)

        @plsc.parallel_loop(0, CHUNK, unroll=8)
        def zero_row(e):
            p_v[0, e] = jnp.zeros((16,), jnp.float32)

        for c in range(rows_pt // CHUNK):
            pltpu.sync_copy(p_v.at[0],
                            den_sp.at[pl.ds(sid * rows_pt + c * CHUNK,
                                            CHUNK)])
        pltpu.sync_copy(b16_h, b_v)
        pltpu.sync_copy(srci_h.at[pl.ds(base_c, N0)], si_v)
        pltpu.sync_copy(dsti_h.at[pl.ds(base_c, N0)], di_v)
        plsc.subcore_barrier()
        bb = b_v[...]

        def fire(i, buf):
            pltpu.async_copy(tsrc_h.at[si_v.at[i]], s_v.at[buf], gsem)
            pltpu.async_copy(tdst_h.at[di_v.at[i]], d_v.at[buf], gsem)

        fire(0, 0)

        def chunk_body(i, carry):
            cur = lax.rem(i, 2)
            nxt = lax.rem(i + 1, 2)
            base = (base_c + i) * CHUNK

            @pl.when(i > 0)
            def _():
                pltpu.make_async_copy(
                    p_v.at[nxt], den_sp.at[di_v.at[i - 1]], ssem).wait()
                pltpu.make_async_copy(
                    p_v.at[nxt], p_h.at[pl.ds(base - CHUNK, CHUNK)],
                    psem).wait()

            @pl.when(i + 1 < n_my)
            def _():
                fire(i + 1, nxt)

            pltpu.make_async_copy(tsrc_h.at[si_v.at[i]], s_v.at[cur],
                                  gsem).wait()
            pltpu.make_async_copy(tdst_h.at[di_v.at[i]], d_v.at[cur],
                                  gsem).wait()

            @plsc.parallel_loop(0, CHUNK, unroll=8)
            def edge_body(e):
                ev = s_v[cur, e] + d_v[cur, e]
                ev = jnp.maximum(ev, 0.0) + 0.2 * jnp.minimum(ev, 0.0)
                p_v[cur, e] = jnp.exp(ev - bb)
            pltpu.async_copy(p_v.at[cur], den_sp.at[di_v.at[i]], ssem,
                             add=True)
            pltpu.async_copy(p_v.at[cur], p_h.at[pl.ds(base, CHUNK)], psem)
            return carry

        lax.fori_loop(0, n_my, chunk_body, 0)
        last = lax.rem(n_my - 1, 2)
        pltpu.make_async_copy(
            p_v.at[last], den_sp.at[di_v.at[n_my - 1]], ssem).wait()
        pltpu.make_async_copy(
            p_v.at[last],
            p_h.at[pl.ds((base_c + n_my - 1) * CHUNK, CHUNK)], psem).wait()
        plsc.subcore_barrier()
        pltpu.sync_copy(den_sp.at[pl.ds(sid * rows_pt, rows_pt)],
                        den_h.at[cid, pl.ds(sid * rows_pt, rows_pt)])

    return k(tsrc, tdst, b16, srci, dsti)


def _sc_messages(h, den0, den1, p, srci, dsti, NP, EP, n):
    n_pair = (EP // CHUNK) // NS
    N0, N1 = _core_split(n_pair)
    NPS = _round_up(n + 1, NS)
    rows_pt = NPS // NS
    mesh = plsc.VectorSubcoreMesh(core_axis_name="c", subcore_axis_name="s",
                                  num_cores=NC, num_subcores=NS)

    @functools.partial(
        pl.kernel, mesh=mesh,
        compiler_params=pltpu.CompilerParams(use_tc_tiling_on_sc=False),
        out_type=jax.ShapeDtypeStruct((NC, NP, HC), jnp.float32),
        scratch_types=[
            pltpu.VMEM((3, CHUNK), jnp.int32),
            pltpu.VMEM((3, CHUNK), jnp.int32),
            pltpu.VMEM((2, CHUNK, 16), jnp.float32),
            pltpu.VMEM((2, CHUNK, 16), jnp.float32),
            pltpu.VMEM((2, CHUNK, 16), jnp.float32),
            pltpu.VMEM((2, CHUNK, HC), jnp.float32),
            pltpu.VMEM_SHARED((NPS, HC), jnp.float32),
            pltpu.SemaphoreType.DMA,
            pltpu.SemaphoreType.DMA,
            pltpu.SemaphoreType.DMA,
        ])
    def k(h_h, d0_h, d1_h, p_h, srci_h, dsti_h, out_h,
          si_v, di_v, pa_v, d0_v, d1_v, hs_v, out_sp, gsem, ssem, isem):
        cid = lax.axis_index("c")
        sid = lax.axis_index("s")
        n_my = jnp.where(cid == 0, N0, N1)
        base_w = jnp.where(cid == 0, sid * N0, NS * N0 + sid * N1)

        @plsc.parallel_loop(0, CHUNK, unroll=4)
        def zero_row(e):
            for j in range(H):
                hs_v[0, e, pl.ds(j * 16, 16)] = jnp.zeros((16,), jnp.float32)

        for c in range(rows_pt // CHUNK):
            pltpu.sync_copy(hs_v.at[0],
                            out_sp.at[pl.ds(sid * rows_pt + c * CHUNK,
                                            CHUNK)])
        rem = rows_pt % CHUNK
        if rem:
            pltpu.sync_copy(
                hs_v.at[0, pl.ds(0, rem)],
                out_sp.at[pl.ds(sid * rows_pt + rows_pt - rem, rem)])
        plsc.subcore_barrier()

        def load_idx(i, slot):
            pltpu.async_copy(srci_h.at[pl.ds(base_w + i, 1)],
                             si_v.at[pl.ds(slot, 1)], isem)
            pltpu.async_copy(dsti_h.at[pl.ds(base_w + i, 1)],
                             di_v.at[pl.ds(slot, 1)], isem)

        def fire(i, buf, slot):
            pltpu.make_async_copy(srci_h.at[pl.ds(base_w + i, 1)],
                                  si_v.at[pl.ds(slot, 1)], isem).wait()
            pltpu.make_async_copy(dsti_h.at[pl.ds(base_w + i, 1)],
                                  di_v.at[pl.ds(slot, 1)], isem).wait()
            base = (base_w + i) * CHUNK
            pltpu.async_copy(d0_h.at[di_v.at[slot]], d0_v.at[buf], gsem)
            pltpu.async_copy(d1_h.at[di_v.at[slot]], d1_v.at[buf], gsem)
            pltpu.async_copy(p_h.at[pl.ds(base, CHUNK)], pa_v.at[buf], gsem)
            pltpu.async_copy(h_h.at[si_v.at[slot]], hs_v.at[buf], gsem)

        load_idx(0, 0)
        fire(0, 0, 0)
        load_idx(1, 1)

        def chunk_body(i, carry):
            cur = lax.rem(i, 2)
            nxt = lax.rem(i + 1, 2)
            slot = lax.rem(i, 3)
            slot1 = lax.rem(i + 1, 3)
            slot2 = lax.rem(i + 2, 3)
            base = (base_w + i) * CHUNK

            @pl.when(i > 0)
            def _():
                pltpu.make_async_copy(
                    hs_v.at[nxt], out_sp.at[di_v.at[slot2]], ssem).wait()

            @pl.when(i + 1 < n_my)
            def _():
                fire(i + 1, nxt, slot1)

            @pl.when(i + 2 < n_my)
            def _():
                load_idx(i + 2, slot2)

            pltpu.make_async_copy(d0_h.at[di_v.at[slot]], d0_v.at[cur],
                                  gsem).wait()
            pltpu.make_async_copy(d1_h.at[di_v.at[slot]], d1_v.at[cur],
                                  gsem).wait()
            pltpu.make_async_copy(p_h.at[pl.ds(base, CHUNK)], pa_v.at[cur],
                                  gsem).wait()
            pltpu.make_async_copy(h_h.at[si_v.at[slot]], hs_v.at[cur],
                                  gsem).wait()

            @plsc.parallel_loop(0, CHUNK, unroll=4)
            def edge_body(e):
                al = pa_v[cur, e] / (d0_v[cur, e] + d1_v[cur, e] + 1e-16)
                for j in range(H):
                    sc = al[j]
                    hs_v[cur, e, pl.ds(j * 16, 16)] = (
                        hs_v[cur, e, pl.ds(j * 16, 16)] * sc)
            pltpu.async_copy(hs_v.at[cur], out_sp.at[di_v.at[slot]], ssem,
                             add=True)
            return carry

        lax.fori_loop(0, n_my, chunk_body, 0)
        last = lax.rem(n_my - 1, 2)
        pltpu.make_async_copy(
            hs_v.at[last],
            out_sp.at[di_v.at[lax.rem(n_my - 1, 3)]], ssem).wait()
        plsc.subcore_barrier()
        pltpu.sync_copy(out_sp.at[pl.ds(sid * rows_pt, rows_pt)],
                        out_h.at[cid, pl.ds(sid * rows_pt, rows_pt)])

    return k(h, den0, den1, p, srci, dsti)


def _mmat(a):
    af = a.reshape(HC)
    ci = jnp.arange(HC, dtype=jnp.int32) // C
    jj = jnp.arange(16, dtype=jnp.int32) % H
    mask = (ci[:, None] == jj[None, :]).astype(jnp.float32)
    return af[:, None] * mask


def kernel(x, edge_index, batch, W0, a_src0, a_dst0, b0, W1, a_src1, a_dst1,
           b1, W2, a_src2, a_dst2, b2, Wp, bp):
    n = x.shape[0]
    e = edge_index.shape[1]
    NP = _round_up(n + 1, BLK)
    ep_raw = e + n
    per_tile = _round_up(-(-ep_raw // NW), CHUNK)
    EP = per_tile * NW

    loop = jnp.arange(n, dtype=jnp.int32)
    srci = jnp.pad(jnp.concatenate(
        [edge_index[0], loop,
         jnp.zeros((EP - ep_raw,), jnp.int32)]).reshape(EP // CHUNK, CHUNK),
        ((0, 32), (0, 0)))
    dsti = jnp.pad(jnp.concatenate(
        [edge_index[1], loop,
         jnp.full((EP - ep_raw,), n, jnp.int32)]).reshape(EP // CHUNK, CHUNK),
        ((0, 32), (0, 0)))

    xp = jnp.pad(x, ((0, NP - n), (0, 0)))

    Ws = [W0, W1, W2]
    Ms = [(_mmat(a_src0), _mmat(a_dst0)),
          (_mmat(a_src1), _mmat(a_dst1)),
          (_mmat(a_src2), _mmat(a_dst2))]
    bs = [b0.reshape(1, HC), b1.reshape(1, HC), b2.reshape(1, HC)]

    part0 = part1 = None
    for l in range(3):
        if l == 0:
            hmat, ts, td, mx = _prep_first(xp, Ws[0], Ms[0][0], Ms[0][1], NP)
        else:
            hmat, ts, td, mx = _prep_next(part0, part1, bs[l - 1], Ws[l],
                                          Ms[l][0], Ms[l][1], NP, n)
        b16 = mx[0] + mx[1]
        p, den2 = _sc_edge_logits(ts, td, b16, srci, dsti, NP, EP)
        out2 = _sc_messages(hmat, den2[0], den2[1], p, srci, dsti,
                            NP, EP, n)
        part0, part1 = out2[0], out2[1]

    return _final_proj(part0, part1, bs[2], Wp, bp.reshape(1, 3), NP, n)
